# Initial kernel scaffold; baseline (speedup 1.0000x reference)
import jax
import jax.numpy as jnp
from jax.experimental import pallas as pl


def kernel(node_features, edge_index, edge_attr, pipeline_state, register_pressure, ready_mask, scheduled_mask, enc_W1, enc_b1, enc_W2, enc_b2, gat_W, gat_att_src, gat_att_dst, gat_bias, ln_g, ln_b, pip_W1, pip_b1, pip_W2, pip_b2):
    raise NotImplementedError("write your pallas kernel here")


# speedup vs baseline: 39.3596x; 39.3596x over previous
import functools

import jax
import jax.numpy as jnp
import numpy as np
from jax import lax
from jax.experimental import pallas as pl
from jax.experimental.pallas import tpu as pltpu
from jax.experimental.pallas import tpu_sc as plsc

N = 50000
E = 800000
HID = 64
HEADS = 4
FH = 16
NFEAT = 48
NLAYERS = 3

NP = 50176
DUMMY = 50000
NB = 49
BR = 1024
CH = 128
EPAD = 851968
NSTRIPE = NP // 16


def _enc_body(nf, w1, b1, w2, b2, out):
    h1 = jnp.maximum(jnp.dot(nf[...], w1[...], preferred_element_type=jnp.float32)
                     + b1[...], 0.0)
    out[...] = jnp.dot(h1, w2[...], preferred_element_type=jnp.float32) + b2[...]


def _encode(nf_pad, w1, b1, w2, b2):
    return pl.pallas_call(
        _enc_body,
        grid=(NB,),
        in_specs=[
            pl.BlockSpec((BR, NFEAT), lambda i: (i, 0)),
            pl.BlockSpec((NFEAT, HID), lambda i: (0, 0)),
            pl.BlockSpec((1, HID), lambda i: (0, 0)),
            pl.BlockSpec((HID, HID), lambda i: (0, 0)),
            pl.BlockSpec((1, HID), lambda i: (0, 0)),
        ],
        out_specs=pl.BlockSpec((BR, HID), lambda i: (i, 0)),
        out_shape=jax.ShapeDtypeStruct((NP, HID), jnp.float32),
    )(nf_pad, w1, b1, w2, b2)


def _proj_body(h, w, a, x2, asd):
    x = jnp.dot(h[...], w[...], preferred_element_type=jnp.float32)
    asd[...] = jnp.dot(x, a[...], preferred_element_type=jnp.float32)
    j = pl.program_id(1)
    x2[...] = jnp.where(j == 0, x[:, :32], x[:, 32:])


def _project(h, w, acat):
    return pl.pallas_call(
        _proj_body,
        grid=(NB, 2),
        in_specs=[
            pl.BlockSpec((BR, HID), lambda i, j: (i, 0)),
            pl.BlockSpec((HID, HID), lambda i, j: (0, 0)),
            pl.BlockSpec((HID, 2 * HEADS), lambda i, j: (0, 0)),
        ],
        out_specs=[
            pl.BlockSpec((BR, 32), lambda i, j: (j * NB + i, 0)),
            pl.BlockSpec((BR, 2 * HEADS), lambda i, j: (i, 0)),
        ],
        out_shape=[
            jax.ShapeDtypeStruct((2 * NP, 32), jnp.float32),
            jax.ShapeDtypeStruct((NP, 2 * HEADS), jnp.float32),
        ],
    )(h, w, acat)


def _post_body(hres, olo, ohi, d0, d1, sel, bias, g, b, out):
    den = d0[...][0][:, :HEADS] + d1[...][0][:, :HEADS]
    inv = 1.0 / (den + 1e-16)
    inv64 = jnp.dot(inv, sel[...], preferred_element_type=jnp.float32)
    agg = jnp.concatenate([olo[...][0], ohi[...][0]], axis=-1)
    y = agg * inv64 + bias[...] + hres[...]
    m = jnp.mean(y, axis=-1, keepdims=True)
    yc = y - m
    var = jnp.mean(yc * yc, axis=-1, keepdims=True)
    out[...] = jnp.maximum(yc * lax.rsqrt(var + 1e-5) * g[...] + b[...], 0.0)


def _postprocess(hres, out2, denp, sel, bias, g, b):
    return pl.pallas_call(
        _post_body,
        grid=(NB,),
        in_specs=[
            pl.BlockSpec((BR, HID), lambda i: (i, 0)),
            pl.BlockSpec((1, BR, 32), lambda i: (0, i, 0)),
            pl.BlockSpec((1, BR, 32), lambda i: (1, i, 0)),
            pl.BlockSpec((1, BR, 8), lambda i: (0, i, 0)),
            pl.BlockSpec((1, BR, 8), lambda i: (1, i, 0)),
            pl.BlockSpec((HEADS, HID), lambda i: (0, 0)),
            pl.BlockSpec((1, HID), lambda i: (0, 0)),
            pl.BlockSpec((1, HID), lambda i: (0, 0)),
            pl.BlockSpec((1, HID), lambda i: (0, 0)),
        ],
        out_specs=pl.BlockSpec((BR, HID), lambda i: (i, 0)),
        out_shape=jax.ShapeDtypeStruct((NP, HID), jnp.float32),
    )(hres, out2, out2, denp, denp, sel, bias, g, b)


def _mlp_body(pf, w1, b1, w2, b2, out):
    h1 = jnp.maximum(jnp.dot(pf[...], w1[...], preferred_element_type=jnp.float32)
                     + b1[...], 0.0)
    out[...] = jnp.dot(h1, w2[...], preferred_element_type=jnp.float32) + b2[...]


def _pipeline_mlp(pf_pad, w1p, b1, w2, b2):
    return pl.pallas_call(
        _mlp_body,
        grid=(1,),
        in_specs=[
            pl.BlockSpec((8, 16), lambda i: (0, 0)),
            pl.BlockSpec((16, HID), lambda i: (0, 0)),
            pl.BlockSpec((1, HID), lambda i: (0, 0)),
            pl.BlockSpec((HID, HID), lambda i: (0, 0)),
            pl.BlockSpec((1, HID), lambda i: (0, 0)),
        ],
        out_specs=pl.BlockSpec((8, HID), lambda i: (0, 0)),
        out_shape=jax.ShapeDtypeStruct((8, HID), jnp.float32),
    )(pf_pad, w1p, b1, w2, b2)


def _sc_edge_body(src_h, dst_h, asd_h, x2_h, z32_h,
                  out2_h,
                  idxs, idxd, bufs, bufd, exb, xbuf, acc,
                  out_sh, sem1, sem2, sem3):
    c = lax.axis_index("c")
    s = lax.axis_index("s")
    lo = s * NSTRIPE
    pltpu.sync_copy(z32_h.at[pl.ds(lo, NSTRIPE)], out_sh.at[pl.ds(lo, NSTRIPE)])
    plsc.subcore_barrier()

    iota = lax.broadcasted_iota(jnp.int32, (16,), 0)
    ebase = s * (EPAD // 16)
    nchunks = EPAD // 16 // CH
    xoff = c * NP
    cb = 2 * c

    def chunk(i, carry):
        e0 = ebase + i * CH
        pltpu.sync_copy(src_h.at[pl.ds(e0, CH)], idxs)
        pltpu.sync_copy(dst_h.at[pl.ds(e0, CH)], idxd)
        ga = pltpu.async_copy(asd_h.at[idxs], bufs, sem1)
        gb = pltpu.async_copy(asd_h.at[idxd], bufd, sem2)
        ga.wait()
        gb.wait()

        def lane(j, cr):
            p = j * 16 + iota
            r = p // 4
            col = p % 4
            vs = plsc.load_gather(bufs, [r, col])
            vd = plsc.load_gather(bufd, [r, col + 4])
            al = vs + vd
            al = jnp.where(al >= 0.0, al, al * 0.2)
            plsc.store_scatter(exb, [r, col], jnp.exp(al))
            return cr

        lax.fori_loop(0, (CH * 4) // 16, lane, 0, unroll=4)

        def adj(j, cr):
            sl = pl.ds(j * 16, 16)
            idxs[sl] = idxs[sl] + xoff
            return cr

        lax.fori_loop(0, CH // 16, adj, 0, unroll=8)
        pltpu.async_copy(x2_h.at[idxs], xbuf, sem3).wait()

        def edge(e, cr):
            ev = jnp.broadcast_to(e, (16,))
            c0 = plsc.load_gather(exb, [ev, jnp.broadcast_to(cb, (16,))])
            c1 = plsc.load_gather(exb, [ev, jnp.broadcast_to(cb + 1, (16,))])
            acc[e, pl.ds(0, 16)] = xbuf[e, pl.ds(0, 16)] * c0
            acc[e, pl.ds(16, 16)] = xbuf[e, pl.ds(16, 16)] * c1
            return cr

        lax.fori_loop(0, CH, edge, 0, unroll=4)
        pltpu.sync_copy(acc, out_sh.at[idxd], add=True)
        return carry

    lax.fori_loop(0, nchunks, chunk, 0)
    plsc.subcore_barrier()
    pltpu.sync_copy(out_sh.at[pl.ds(lo, NSTRIPE)],
                    out2_h.at[c, pl.ds(lo, NSTRIPE)])


@functools.partial(
    pl.kernel,
    mesh=plsc.VectorSubcoreMesh(core_axis_name="c", subcore_axis_name="s"),
    compiler_params=pltpu.CompilerParams(
        use_tc_tiling_on_sc=False, needs_layout_passes=False),
    out_type=[
        jax.ShapeDtypeStruct((2, NP, 32), jnp.float32),
    ],
    scratch_types=[
        pltpu.VMEM((CH,), jnp.int32),
        pltpu.VMEM((CH,), jnp.int32),
        pltpu.VMEM((CH, 2 * HEADS), jnp.float32),
        pltpu.VMEM((CH, 2 * HEADS), jnp.float32),
        pltpu.VMEM((CH, HEADS), jnp.float32),
        pltpu.VMEM((CH, 32), jnp.float32),
        pltpu.VMEM((CH, 32), jnp.float32),
        pltpu.VMEM_SHARED((NP, 32), jnp.float32),
        pltpu.SemaphoreType.DMA,
        pltpu.SemaphoreType.DMA,
        pltpu.SemaphoreType.DMA,
    ],
)
def _sc_edge(src_h, dst_h, asd_h, x2_h, z32_h, out2_h,
             idxs, idxd, bufs, bufd, exb, xbuf, acc,
             out_sh, sem1, sem2, sem3):
    _sc_edge_body(src_h, dst_h, asd_h, x2_h, z32_h, out2_h,
                  idxs, idxd, bufs, bufd, exb, xbuf, acc,
                  out_sh, sem1, sem2, sem3)


def _sc_den_body(src_h, dst_h, asd_h, z8_h, den_h,
                 idxs, idxd, bufs, bufd, exb, den_sh, sem1, sem2):
    c = lax.axis_index("c")
    s = lax.axis_index("s")
    lo = s * NSTRIPE
    pltpu.sync_copy(z8_h.at[pl.ds(lo, NSTRIPE)], den_sh.at[pl.ds(lo, NSTRIPE)])
    plsc.subcore_barrier()
    iota = lax.broadcasted_iota(jnp.int32, (16,), 0)
    ebase = (c * 16 + s) * (EPAD // 32)
    nchunks = EPAD // 32 // CH

    def chunk(i, carry):
        e0 = ebase + i * CH
        pltpu.sync_copy(src_h.at[pl.ds(e0, CH)], idxs)
        pltpu.sync_copy(dst_h.at[pl.ds(e0, CH)], idxd)
        ga = pltpu.async_copy(asd_h.at[idxs], bufs, sem1)
        gb = pltpu.async_copy(asd_h.at[idxd], bufd, sem2)
        ga.wait()
        gb.wait()

        def lane(j, cr):
            p = j * 16 + iota
            r = p // 8
            col = p % 8
            hd = col % 4
            vs = plsc.load_gather(bufs, [r, hd])
            vd = plsc.load_gather(bufd, [r, hd + 4])
            al = vs + vd
            al = jnp.where(al >= 0.0, al, al * 0.2)
            plsc.store_scatter(exb, [r, col], jnp.exp(al))
            return cr

        lax.fori_loop(0, (CH * 8) // 16, lane, 0, unroll=4)
        pltpu.sync_copy(exb, den_sh.at[idxd], add=True)
        return carry

    lax.fori_loop(0, nchunks, chunk, 0)
    plsc.subcore_barrier()
    pltpu.sync_copy(den_sh.at[pl.ds(lo, NSTRIPE)],
                    den_h.at[c, pl.ds(lo, NSTRIPE)])


@functools.partial(
    pl.kernel,
    mesh=plsc.VectorSubcoreMesh(core_axis_name="c", subcore_axis_name="s"),
    compiler_params=pltpu.CompilerParams(
        use_tc_tiling_on_sc=False, needs_layout_passes=False),
    out_type=[jax.ShapeDtypeStruct((2, NP, 8), jnp.float32)],
    scratch_types=[
        pltpu.VMEM((CH,), jnp.int32),
        pltpu.VMEM((CH,), jnp.int32),
        pltpu.VMEM((CH, 2 * HEADS), jnp.float32),
        pltpu.VMEM((CH, 2 * HEADS), jnp.float32),
        pltpu.VMEM((CH, 8), jnp.float32),
        pltpu.VMEM_SHARED((NP, 8), jnp.float32),
        pltpu.SemaphoreType.DMA,
        pltpu.SemaphoreType.DMA,
    ],
)
def _sc_den(src_h, dst_h, asd_h, z8_h, den_h,
            idxs, idxd, bufs, bufd, exb, den_sh, sem1, sem2):
    _sc_den_body(src_h, dst_h, asd_h, z8_h, den_h,
                 idxs, idxd, bufs, bufd, exb, den_sh, sem1, sem2)


def kernel(node_features, edge_index, edge_attr, pipeline_state,
           register_pressure, ready_mask, scheduled_mask,
           enc_W1, enc_b1, enc_W2, enc_b2,
           gat_W, gat_att_src, gat_att_dst, gat_bias,
           ln_g, ln_b,
           pip_W1, pip_b1, pip_W2, pip_b2):
    f32 = jnp.float32
    nf_pad = jnp.pad(node_features, ((0, NP - N), (0, 0)))

    loops = jnp.arange(N, dtype=edge_index.dtype)
    pad_e = jnp.full((EPAD - E - N,), DUMMY, dtype=edge_index.dtype)
    src = jnp.concatenate([edge_index[0], loops, pad_e])
    dst = jnp.concatenate([edge_index[1], loops, pad_e])

    z8 = jnp.zeros((NP, 8), f32)
    z32 = jnp.zeros((NP, 32), f32)
    sel = jnp.asarray(np.kron(np.eye(HEADS), np.ones((1, FH))), f32)
    eye = jnp.asarray(np.eye(HEADS), f32)

    b1r = enc_b1.reshape(1, HID)
    b2r = enc_b2.reshape(1, HID)
    h = _encode(nf_pad, enc_W1, b1r, enc_W2, b2r)

    for i in range(NLAYERS):
        a_src = (gat_att_src[i][:, :, None] * eye[:, None, :]).reshape(HID, HEADS)
        a_dst = (gat_att_dst[i][:, :, None] * eye[:, None, :]).reshape(HID, HEADS)
        acat = jnp.concatenate([a_src, a_dst], axis=1)
        x2, asd = _project(h, gat_W[i], acat)
        out2 = _sc_edge(src, dst, asd, x2, z32)[0]
        denp = _sc_den(src, dst, asd, z8)[0]
        h = _postprocess(h, out2, denp, sel,
                         gat_bias[i].reshape(1, HID),
                         ln_g[i].reshape(1, HID), ln_b[i].reshape(1, HID))

    pf = jnp.concatenate([pipeline_state, register_pressure])
    pf_pad = jnp.zeros((8, 16), f32).at[0, :9].set(pf)
    w1p = jnp.pad(pip_W1, ((0, 16 - 9), (0, 0)))
    q = _pipeline_mlp(pf_pad, w1p, pip_b1.reshape(1, HID),
                      pip_W2, pip_b2.reshape(1, HID))
    return (h[:N], q[0])


# speedup vs baseline: 58.6024x; 1.4889x over previous
import functools

import jax
import jax.numpy as jnp
import numpy as np
from jax import lax
from jax.experimental import pallas as pl
from jax.experimental.pallas import tpu as pltpu
from jax.experimental.pallas import tpu_sc as plsc

N = 50000
E = 800000
HID = 64
HEADS = 4
FH = 16
NFEAT = 48
NLAYERS = 3

NP = 50176
DUMMY = 50000
NB = 49
BR = 1024
CH = 128
EPAD = 851968
NSTRIPE = NP // 16


def _enc_body(nf, w1, b1, w2, b2, out):
    h1 = jnp.maximum(jnp.dot(nf[...], w1[...], preferred_element_type=jnp.float32)
                     + b1[...], 0.0)
    out[...] = jnp.dot(h1, w2[...], preferred_element_type=jnp.float32) + b2[...]


def _encode(nf_pad, w1, b1, w2, b2):
    return pl.pallas_call(
        _enc_body,
        grid=(NB,),
        in_specs=[
            pl.BlockSpec((BR, NFEAT), lambda i: (i, 0)),
            pl.BlockSpec((NFEAT, HID), lambda i: (0, 0)),
            pl.BlockSpec((1, HID), lambda i: (0, 0)),
            pl.BlockSpec((HID, HID), lambda i: (0, 0)),
            pl.BlockSpec((1, HID), lambda i: (0, 0)),
        ],
        out_specs=pl.BlockSpec((BR, HID), lambda i: (i, 0)),
        out_shape=jax.ShapeDtypeStruct((NP, HID), jnp.float32),
    )(nf_pad, w1, b1, w2, b2)


def _proj_body(h, w, a, x2, asd):
    x = jnp.dot(h[...], w[...], preferred_element_type=jnp.float32)
    asd[...] = jnp.dot(x, a[...], preferred_element_type=jnp.float32)
    j = pl.program_id(1)
    x2[...] = jnp.where(j == 0, x[:, :32], x[:, 32:])


def _project(h, w, acat):
    return pl.pallas_call(
        _proj_body,
        grid=(NB, 2),
        in_specs=[
            pl.BlockSpec((BR, HID), lambda i, j: (i, 0)),
            pl.BlockSpec((HID, HID), lambda i, j: (0, 0)),
            pl.BlockSpec((HID, 2 * HEADS), lambda i, j: (0, 0)),
        ],
        out_specs=[
            pl.BlockSpec((BR, 32), lambda i, j: (j * NB + i, 0)),
            pl.BlockSpec((BR, 2 * HEADS), lambda i, j: (i, 0)),
        ],
        out_shape=[
            jax.ShapeDtypeStruct((2 * NP, 32), jnp.float32),
            jax.ShapeDtypeStruct((NP, 2 * HEADS), jnp.float32),
        ],
    )(h, w, acat)


def _post_body(hres, olo, ohi, d0, d1, sel, bias, g, b, out):
    den = d0[...][0][:, :HEADS] + d1[...][0][:, :HEADS]
    inv = 1.0 / (den + 1e-16)
    inv64 = jnp.dot(inv, sel[...], preferred_element_type=jnp.float32)
    agg = jnp.concatenate([olo[...][0], ohi[...][0]], axis=-1)
    y = agg * inv64 + bias[...] + hres[...]
    m = jnp.mean(y, axis=-1, keepdims=True)
    yc = y - m
    var = jnp.mean(yc * yc, axis=-1, keepdims=True)
    out[...] = jnp.maximum(yc * lax.rsqrt(var + 1e-5) * g[...] + b[...], 0.0)


def _postprocess(hres, out2, denp, sel, bias, g, b):
    return pl.pallas_call(
        _post_body,
        grid=(NB,),
        in_specs=[
            pl.BlockSpec((BR, HID), lambda i: (i, 0)),
            pl.BlockSpec((1, BR, 32), lambda i: (0, i, 0)),
            pl.BlockSpec((1, BR, 32), lambda i: (1, i, 0)),
            pl.BlockSpec((1, BR, 8), lambda i: (0, i, 0)),
            pl.BlockSpec((1, BR, 8), lambda i: (1, i, 0)),
            pl.BlockSpec((HEADS, HID), lambda i: (0, 0)),
            pl.BlockSpec((1, HID), lambda i: (0, 0)),
            pl.BlockSpec((1, HID), lambda i: (0, 0)),
            pl.BlockSpec((1, HID), lambda i: (0, 0)),
        ],
        out_specs=pl.BlockSpec((BR, HID), lambda i: (i, 0)),
        out_shape=jax.ShapeDtypeStruct((NP, HID), jnp.float32),
    )(hres, out2, out2, denp, denp, sel, bias, g, b)


def _mlp_body(pf, w1, b1, w2, b2, out):
    h1 = jnp.maximum(jnp.dot(pf[...], w1[...], preferred_element_type=jnp.float32)
                     + b1[...], 0.0)
    out[...] = jnp.dot(h1, w2[...], preferred_element_type=jnp.float32) + b2[...]


def _pipeline_mlp(pf_pad, w1p, b1, w2, b2):
    return pl.pallas_call(
        _mlp_body,
        grid=(1,),
        in_specs=[
            pl.BlockSpec((8, 16), lambda i: (0, 0)),
            pl.BlockSpec((16, HID), lambda i: (0, 0)),
            pl.BlockSpec((1, HID), lambda i: (0, 0)),
            pl.BlockSpec((HID, HID), lambda i: (0, 0)),
            pl.BlockSpec((1, HID), lambda i: (0, 0)),
        ],
        out_specs=pl.BlockSpec((8, HID), lambda i: (0, 0)),
        out_shape=jax.ShapeDtypeStruct((8, HID), jnp.float32),
    )(pf_pad, w1p, b1, w2, b2)


GB = 2
NROW = EPAD // CH
NG_E = EPAD // 16 // (GB * CH)
NG_D = EPAD // 32 // (GB * CH)


def _sc_edge_body(src2_h, dst2_h, asd_h, x2_h, z32_h, out2_h,
                  idxs2, idxx2, idxd2, bufs3, bufd3, exb3, xbuf3, acc3,
                  out_sh, sem1, sem3, sem4):
    c = lax.axis_index("c")
    s = lax.axis_index("s")
    lo = s * NSTRIPE
    pltpu.sync_copy(z32_h.at[pl.ds(lo, NSTRIPE)], out_sh.at[pl.ds(lo, NSTRIPE)])
    plsc.subcore_barrier()

    iota = lax.broadcasted_iota(jnp.int32, (16,), 0)
    rbase = s * (NROW // 16)
    xoff = c * NP
    cb = 2 * c

    def group(g, carry):
        r0 = rbase + g * GB
        pltpu.sync_copy(src2_h.at[pl.ds(r0, GB)], idxs2)
        pltpu.sync_copy(dst2_h.at[pl.ds(r0, GB)], idxd2)

        def adj(j, cr):
            b = j // (CH // 16)
            k = j % (CH // 16)
            sl = pl.ds(k * 16, 16)
            idxx2[b, sl] = idxs2[b, sl] + xoff
            return cr

        lax.fori_loop(0, GB * (CH // 16), adj, 0, unroll=4)

        gs = [pltpu.async_copy(asd_h.at[idxs2.at[b]], bufs3.at[b], sem1)
              for b in range(GB)]
        gd = [pltpu.async_copy(asd_h.at[idxd2.at[b]], bufd3.at[b], sem1)
              for b in range(GB)]
        gx = [pltpu.async_copy(x2_h.at[idxx2.at[b]], xbuf3.at[b], sem3)
              for b in range(GB)]
        for d in gs:
            d.wait()
        for d in gd:
            d.wait()

        def lane(j, cr):
            b = j // ((CH * 4) // 16)
            p = (j % ((CH * 4) // 16)) * 16 + iota
            r = p // 4
            col = p % 4
            bv = jnp.broadcast_to(b, (16,))
            vs = plsc.load_gather(bufs3, [bv, r, col])
            vd = plsc.load_gather(bufd3, [bv, r, col + 4])
            al = vs + vd
            al = jnp.where(al >= 0.0, al, al * 0.2)
            plsc.store_scatter(exb3, [bv, r, col], jnp.exp(al))
            return cr

        lax.fori_loop(0, GB * ((CH * 4) // 16), lane, 0, unroll=4)
        for d in gx:
            d.wait()

        def edge(j, cr):
            b = j // CH
            e = j % CH
            bv = jnp.broadcast_to(b, (16,))
            ev = jnp.broadcast_to(e, (16,))
            c0 = plsc.load_gather(exb3, [bv, ev, jnp.broadcast_to(cb, (16,))])
            c1 = plsc.load_gather(exb3, [bv, ev, jnp.broadcast_to(cb + 1, (16,))])
            acc3[b, e, pl.ds(0, 16)] = xbuf3[b, e, pl.ds(0, 16)] * c0
            acc3[b, e, pl.ds(16, 16)] = xbuf3[b, e, pl.ds(16, 16)] * c1
            return cr

        lax.fori_loop(0, GB * CH, edge, 0, unroll=4)
        sc = [pltpu.async_copy(acc3.at[b], out_sh.at[idxd2.at[b]], sem4, add=True)
              for b in range(GB)]
        for d in sc:
            d.wait()
        return carry

    lax.fori_loop(0, NG_E, group, 0)
    plsc.subcore_barrier()
    pltpu.sync_copy(out_sh.at[pl.ds(lo, NSTRIPE)],
                    out2_h.at[c, pl.ds(lo, NSTRIPE)])


@functools.partial(
    pl.kernel,
    mesh=plsc.VectorSubcoreMesh(core_axis_name="c", subcore_axis_name="s"),
    compiler_params=pltpu.CompilerParams(
        use_tc_tiling_on_sc=False, needs_layout_passes=False),
    out_type=[
        jax.ShapeDtypeStruct((2, NP, 32), jnp.float32),
    ],
    scratch_types=[
        pltpu.VMEM((GB, CH), jnp.int32),
        pltpu.VMEM((GB, CH), jnp.int32),
        pltpu.VMEM((GB, CH), jnp.int32),
        pltpu.VMEM((GB, CH, 2 * HEADS), jnp.float32),
        pltpu.VMEM((GB, CH, 2 * HEADS), jnp.float32),
        pltpu.VMEM((GB, CH, HEADS), jnp.float32),
        pltpu.VMEM((GB, CH, 32), jnp.float32),
        pltpu.VMEM((GB, CH, 32), jnp.float32),
        pltpu.VMEM_SHARED((NP, 32), jnp.float32),
        pltpu.SemaphoreType.DMA,
        pltpu.SemaphoreType.DMA,
        pltpu.SemaphoreType.DMA,
    ],
)
def _sc_edge(src2_h, dst2_h, asd_h, x2_h, z32_h, out2_h,
             idxs2, idxx2, idxd2, bufs3, bufd3, exb3, xbuf3, acc3,
             out_sh, sem1, sem3, sem4):
    _sc_edge_body(src2_h, dst2_h, asd_h, x2_h, z32_h, out2_h,
                  idxs2, idxx2, idxd2, bufs3, bufd3, exb3, xbuf3, acc3,
                  out_sh, sem1, sem3, sem4)


def _sc_den_body(src2_h, dst2_h, asd_h, z8_h, den_h,
                 idxs2, idxd2, bufs3, bufd3, exb3, den_sh, sem1, sem4):
    c = lax.axis_index("c")
    s = lax.axis_index("s")
    lo = s * NSTRIPE
    pltpu.sync_copy(z8_h.at[pl.ds(lo, NSTRIPE)], den_sh.at[pl.ds(lo, NSTRIPE)])
    plsc.subcore_barrier()
    iota = lax.broadcasted_iota(jnp.int32, (16,), 0)
    rbase = (c * 16 + s) * (NROW // 32)

    def group(g, carry):
        r0 = rbase + g * GB
        pltpu.sync_copy(src2_h.at[pl.ds(r0, GB)], idxs2)
        pltpu.sync_copy(dst2_h.at[pl.ds(r0, GB)], idxd2)
        gs = [pltpu.async_copy(asd_h.at[idxs2.at[b]], bufs3.at[b], sem1)
              for b in range(GB)]
        gd = [pltpu.async_copy(asd_h.at[idxd2.at[b]], bufd3.at[b], sem1)
              for b in range(GB)]
        for d in gs:
            d.wait()
        for d in gd:
            d.wait()

        def lane(j, cr):
            b = j // ((CH * 8) // 16)
            p = (j % ((CH * 8) // 16)) * 16 + iota
            r = p // 8
            col = p % 8
            hd = col % 4
            bv = jnp.broadcast_to(b, (16,))
            vs = plsc.load_gather(bufs3, [bv, r, hd])
            vd = plsc.load_gather(bufd3, [bv, r, hd + 4])
            al = vs + vd
            al = jnp.where(al >= 0.0, al, al * 0.2)
            plsc.store_scatter(exb3, [bv, r, col], jnp.exp(al))
            return cr

        lax.fori_loop(0, GB * ((CH * 8) // 16), lane, 0, unroll=4)
        sc = [pltpu.async_copy(exb3.at[b], den_sh.at[idxd2.at[b]], sem4, add=True)
              for b in range(GB)]
        for d in sc:
            d.wait()
        return carry

    lax.fori_loop(0, NG_D, group, 0)
    plsc.subcore_barrier()
    pltpu.sync_copy(den_sh.at[pl.ds(lo, NSTRIPE)],
                    den_h.at[c, pl.ds(lo, NSTRIPE)])


@functools.partial(
    pl.kernel,
    mesh=plsc.VectorSubcoreMesh(core_axis_name="c", subcore_axis_name="s"),
    compiler_params=pltpu.CompilerParams(
        use_tc_tiling_on_sc=False, needs_layout_passes=False),
    out_type=[jax.ShapeDtypeStruct((2, NP, 8), jnp.float32)],
    scratch_types=[
        pltpu.VMEM((GB, CH), jnp.int32),
        pltpu.VMEM((GB, CH), jnp.int32),
        pltpu.VMEM((GB, CH, 2 * HEADS), jnp.float32),
        pltpu.VMEM((GB, CH, 2 * HEADS), jnp.float32),
        pltpu.VMEM((GB, CH, 8), jnp.float32),
        pltpu.VMEM_SHARED((NP, 8), jnp.float32),
        pltpu.SemaphoreType.DMA,
        pltpu.SemaphoreType.DMA,
    ],
)
def _sc_den(src2_h, dst2_h, asd_h, z8_h, den_h,
            idxs2, idxd2, bufs3, bufd3, exb3, den_sh, sem1, sem4):
    _sc_den_body(src2_h, dst2_h, asd_h, z8_h, den_h,
                 idxs2, idxd2, bufs3, bufd3, exb3, den_sh, sem1, sem4)


def kernel(node_features, edge_index, edge_attr, pipeline_state,
           register_pressure, ready_mask, scheduled_mask,
           enc_W1, enc_b1, enc_W2, enc_b2,
           gat_W, gat_att_src, gat_att_dst, gat_bias,
           ln_g, ln_b,
           pip_W1, pip_b1, pip_W2, pip_b2):
    f32 = jnp.float32
    nf_pad = jnp.pad(node_features, ((0, NP - N), (0, 0)))

    loops = jnp.arange(N, dtype=edge_index.dtype)
    pad_e = jnp.full((EPAD - E - N,), DUMMY, dtype=edge_index.dtype)
    src = jnp.concatenate([edge_index[0], loops, pad_e]).reshape(NROW, CH)
    dst = jnp.concatenate([edge_index[1], loops, pad_e]).reshape(NROW, CH)

    z8 = jnp.zeros((NP, 8), f32)
    z32 = jnp.zeros((NP, 32), f32)
    sel = jnp.asarray(np.kron(np.eye(HEADS), np.ones((1, FH))), f32)
    eye = jnp.asarray(np.eye(HEADS), f32)

    b1r = enc_b1.reshape(1, HID)
    b2r = enc_b2.reshape(1, HID)
    h = _encode(nf_pad, enc_W1, b1r, enc_W2, b2r)

    for i in range(NLAYERS):
        a_src = (gat_att_src[i][:, :, None] * eye[:, None, :]).reshape(HID, HEADS)
        a_dst = (gat_att_dst[i][:, :, None] * eye[:, None, :]).reshape(HID, HEADS)
        acat = jnp.concatenate([a_src, a_dst], axis=1)
        x2, asd = _project(h, gat_W[i], acat)
        out2 = _sc_edge(src, dst, asd, x2, z32)[0]
        denp = _sc_den(src, dst, asd, z8)[0]
        h = _postprocess(h, out2, denp, sel,
                         gat_bias[i].reshape(1, HID),
                         ln_g[i].reshape(1, HID), ln_b[i].reshape(1, HID))

    pf = jnp.concatenate([pipeline_state, register_pressure])
    pf_pad = jnp.zeros((8, 16), f32).at[0, :9].set(pf)
    w1p = jnp.pad(pip_W1, ((0, 16 - 9), (0, 0)))
    q = _pipeline_mlp(pf_pad, w1p, pip_b1.reshape(1, HID),
                      pip_W2, pip_b2.reshape(1, HID))
    return (h[:N], q[0])


# speedup vs baseline: 67.5799x; 1.1532x over previous
import functools

import jax
import jax.numpy as jnp
import numpy as np
from jax import lax
from jax.experimental import pallas as pl
from jax.experimental.pallas import tpu as pltpu
from jax.experimental.pallas import tpu_sc as plsc

N = 50000
E = 800000
HID = 64
HEADS = 4
FH = 16
NFEAT = 48
NLAYERS = 3

NP = 50176
DUMMY = 50000
NB = 49
BR = 1024
CH = 128
EPAD = 851968
NSTRIPE = NP // 16


def _enc_body(nf, w1, b1, w2, b2, out):
    h1 = jnp.maximum(jnp.dot(nf[...], w1[...], preferred_element_type=jnp.float32)
                     + b1[...], 0.0)
    out[...] = jnp.dot(h1, w2[...], preferred_element_type=jnp.float32) + b2[...]


def _encode(nf_pad, w1, b1, w2, b2):
    return pl.pallas_call(
        _enc_body,
        grid=(NB,),
        in_specs=[
            pl.BlockSpec((BR, NFEAT), lambda i: (i, 0)),
            pl.BlockSpec((NFEAT, HID), lambda i: (0, 0)),
            pl.BlockSpec((1, HID), lambda i: (0, 0)),
            pl.BlockSpec((HID, HID), lambda i: (0, 0)),
            pl.BlockSpec((1, HID), lambda i: (0, 0)),
        ],
        out_specs=pl.BlockSpec((BR, HID), lambda i: (i, 0)),
        out_shape=jax.ShapeDtypeStruct((NP, HID), jnp.float32),
    )(nf_pad, w1, b1, w2, b2)


def _proj_body(h, w, a, x2, asd):
    x = jnp.dot(h[...], w[...], preferred_element_type=jnp.float32)
    asd[...] = jnp.dot(x, a[...], preferred_element_type=jnp.float32)
    j = pl.program_id(1)
    x2[...] = jnp.where(j == 0, x[:, :32], x[:, 32:])


def _project(h, w, acat):
    return pl.pallas_call(
        _proj_body,
        grid=(NB, 2),
        in_specs=[
            pl.BlockSpec((BR, HID), lambda i, j: (i, 0)),
            pl.BlockSpec((HID, HID), lambda i, j: (0, 0)),
            pl.BlockSpec((HID, 2 * HEADS), lambda i, j: (0, 0)),
        ],
        out_specs=[
            pl.BlockSpec((BR, 32), lambda i, j: (j * NB + i, 0)),
            pl.BlockSpec((BR, 2 * HEADS), lambda i, j: (i, 0)),
        ],
        out_shape=[
            jax.ShapeDtypeStruct((2 * NP, 32), jnp.float32),
            jax.ShapeDtypeStruct((NP, 2 * HEADS), jnp.float32),
        ],
    )(h, w, acat)


def _post_body(hres, olo, ohi, d0, d1, sel, bias, g, b, out):
    den = d0[...][0][:, :HEADS] + d1[...][0][:, :HEADS]
    inv = 1.0 / (den + 1e-16)
    inv64 = jnp.dot(inv, sel[...], preferred_element_type=jnp.float32)
    agg = jnp.concatenate([olo[...][0], ohi[...][0]], axis=-1)
    y = agg * inv64 + bias[...] + hres[...]
    m = jnp.mean(y, axis=-1, keepdims=True)
    yc = y - m
    var = jnp.mean(yc * yc, axis=-1, keepdims=True)
    out[...] = jnp.maximum(yc * lax.rsqrt(var + 1e-5) * g[...] + b[...], 0.0)


def _postprocess(hres, out2, denp, sel, bias, g, b):
    return pl.pallas_call(
        _post_body,
        grid=(NB,),
        in_specs=[
            pl.BlockSpec((BR, HID), lambda i: (i, 0)),
            pl.BlockSpec((1, BR, 32), lambda i: (0, i, 0)),
            pl.BlockSpec((1, BR, 32), lambda i: (1, i, 0)),
            pl.BlockSpec((1, BR, 8), lambda i: (0, i, 0)),
            pl.BlockSpec((1, BR, 8), lambda i: (1, i, 0)),
            pl.BlockSpec((HEADS, HID), lambda i: (0, 0)),
            pl.BlockSpec((1, HID), lambda i: (0, 0)),
            pl.BlockSpec((1, HID), lambda i: (0, 0)),
            pl.BlockSpec((1, HID), lambda i: (0, 0)),
        ],
        out_specs=pl.BlockSpec((BR, HID), lambda i: (i, 0)),
        out_shape=jax.ShapeDtypeStruct((NP, HID), jnp.float32),
    )(hres, out2, out2, denp, denp, sel, bias, g, b)


def _mlp_body(pf, w1, b1, w2, b2, out):
    h1 = jnp.maximum(jnp.dot(pf[...], w1[...], preferred_element_type=jnp.float32)
                     + b1[...], 0.0)
    out[...] = jnp.dot(h1, w2[...], preferred_element_type=jnp.float32) + b2[...]


def _pipeline_mlp(pf_pad, w1p, b1, w2, b2):
    return pl.pallas_call(
        _mlp_body,
        grid=(1,),
        in_specs=[
            pl.BlockSpec((8, 16), lambda i: (0, 0)),
            pl.BlockSpec((16, HID), lambda i: (0, 0)),
            pl.BlockSpec((1, HID), lambda i: (0, 0)),
            pl.BlockSpec((HID, HID), lambda i: (0, 0)),
            pl.BlockSpec((1, HID), lambda i: (0, 0)),
        ],
        out_specs=pl.BlockSpec((8, HID), lambda i: (0, 0)),
        out_shape=jax.ShapeDtypeStruct((8, HID), jnp.float32),
    )(pf_pad, w1p, b1, w2, b2)


GB = 4
NROW = EPAD // CH
NG_E = EPAD // 16 // (GB * CH)
NG_D = EPAD // 32 // (GB * CH)


def _sc_edge_body(src2_h, dst2_h, asd_h, x2_h, z32_h, out2_h,
                  idxs2, idxd2, bufs3, bufd3, xbuf3,
                  out_sh, sem1, sem3, sem4):
    c = lax.axis_index("c")
    s = lax.axis_index("s")
    lo = s * NSTRIPE
    pltpu.sync_copy(z32_h.at[pl.ds(lo, NSTRIPE)], out_sh.at[pl.ds(lo, NSTRIPE)])
    plsc.subcore_barrier()

    iota = lax.broadcasted_iota(jnp.int32, (16,), 0)
    rbase = s * (NROW // 16)
    xoff = c * NP
    cb = 2 * c

    def group(g, carry):
        r0 = rbase + g * GB
        pltpu.sync_copy(src2_h.at[pl.ds(r0, GB)], idxs2)
        pltpu.sync_copy(dst2_h.at[pl.ds(r0, GB)], idxd2)
        gs = [pltpu.async_copy(asd_h.at[idxs2.at[b]], bufs3.at[b], sem1)
              for b in range(GB)]
        gd = [pltpu.async_copy(asd_h.at[idxd2.at[b]], bufd3.at[b], sem1)
              for b in range(GB)]
        for d in gs:
            d.wait()
        for d in gd:
            d.wait()

        def adj(j, cr):
            b = j // (CH // 16)
            k = j % (CH // 16)
            sl = pl.ds(k * 16, 16)
            idxs2[b, sl] = idxs2[b, sl] + xoff
            return cr

        lax.fori_loop(0, GB * (CH // 16), adj, 0, unroll=4)
        gx = [pltpu.async_copy(x2_h.at[idxs2.at[b]], xbuf3.at[b], sem3)
              for b in range(GB)]

        def lane(j, cr):
            b = j // ((CH * 4) // 16)
            p = (j % ((CH * 4) // 16)) * 16 + iota
            r = p // 4
            col = p % 4
            bv = jnp.broadcast_to(b, (16,))
            vs = plsc.load_gather(bufs3, [bv, r, col])
            vd = plsc.load_gather(bufd3, [bv, r, col + 4])
            al = vs + vd
            al = jnp.where(al >= 0.0, al, al * 0.2)
            plsc.store_scatter(bufd3, [bv, r, col], jnp.exp(al))
            return cr

        lax.fori_loop(0, GB * ((CH * 4) // 16), lane, 0, unroll=4)
        for d in gx:
            d.wait()

        def edge(j, cr):
            b = j // CH
            e = j % CH
            bv = jnp.broadcast_to(b, (16,))
            ev = jnp.broadcast_to(e, (16,))
            c0 = plsc.load_gather(bufd3, [bv, ev, jnp.broadcast_to(cb, (16,))])
            c1 = plsc.load_gather(bufd3, [bv, ev, jnp.broadcast_to(cb + 1, (16,))])
            xbuf3[b, e, pl.ds(0, 16)] = xbuf3[b, e, pl.ds(0, 16)] * c0
            xbuf3[b, e, pl.ds(16, 16)] = xbuf3[b, e, pl.ds(16, 16)] * c1
            return cr

        lax.fori_loop(0, GB * CH, edge, 0, unroll=4)
        sc = [pltpu.async_copy(xbuf3.at[b], out_sh.at[idxd2.at[b]], sem4, add=True)
              for b in range(GB)]
        for d in sc:
            d.wait()
        return carry

    lax.fori_loop(0, NG_E, group, 0)
    plsc.subcore_barrier()
    pltpu.sync_copy(out_sh.at[pl.ds(lo, NSTRIPE)],
                    out2_h.at[c, pl.ds(lo, NSTRIPE)])


@functools.partial(
    pl.kernel,
    mesh=plsc.VectorSubcoreMesh(core_axis_name="c", subcore_axis_name="s"),
    compiler_params=pltpu.CompilerParams(
        use_tc_tiling_on_sc=False, needs_layout_passes=False),
    out_type=[
        jax.ShapeDtypeStruct((2, NP, 32), jnp.float32),
    ],
    scratch_types=[
        pltpu.VMEM((GB, CH), jnp.int32),
        pltpu.VMEM((GB, CH), jnp.int32),
        pltpu.VMEM((GB, CH, 2 * HEADS), jnp.float32),
        pltpu.VMEM((GB, CH, 2 * HEADS), jnp.float32),
        pltpu.VMEM((GB, CH, 32), jnp.float32),
        pltpu.VMEM_SHARED((NP, 32), jnp.float32),
        pltpu.SemaphoreType.DMA,
        pltpu.SemaphoreType.DMA,
        pltpu.SemaphoreType.DMA,
    ],
)
def _sc_edge(src2_h, dst2_h, asd_h, x2_h, z32_h, out2_h,
             idxs2, idxd2, bufs3, bufd3, xbuf3,
             out_sh, sem1, sem3, sem4):
    _sc_edge_body(src2_h, dst2_h, asd_h, x2_h, z32_h, out2_h,
                  idxs2, idxd2, bufs3, bufd3, xbuf3,
                  out_sh, sem1, sem3, sem4)


def _sc_den_body(src2_h, dst2_h, asd_h, z8_h, den_h,
                 idxs2, idxd2, bufs3, bufd3, exb3, den_sh, sem1, sem4):
    c = lax.axis_index("c")
    s = lax.axis_index("s")
    lo = s * NSTRIPE
    pltpu.sync_copy(z8_h.at[pl.ds(lo, NSTRIPE)], den_sh.at[pl.ds(lo, NSTRIPE)])
    plsc.subcore_barrier()
    iota = lax.broadcasted_iota(jnp.int32, (16,), 0)
    rbase = (c * 16 + s) * (NROW // 32)

    def group(g, carry):
        r0 = rbase + g * GB
        pltpu.sync_copy(src2_h.at[pl.ds(r0, GB)], idxs2)
        pltpu.sync_copy(dst2_h.at[pl.ds(r0, GB)], idxd2)
        gs = [pltpu.async_copy(asd_h.at[idxs2.at[b]], bufs3.at[b], sem1)
              for b in range(GB)]
        gd = [pltpu.async_copy(asd_h.at[idxd2.at[b]], bufd3.at[b], sem1)
              for b in range(GB)]
        for d in gs:
            d.wait()
        for d in gd:
            d.wait()

        def lane(j, cr):
            b = j // ((CH * 8) // 16)
            p = (j % ((CH * 8) // 16)) * 16 + iota
            r = p // 8
            col = p % 8
            hd = col % 4
            bv = jnp.broadcast_to(b, (16,))
            vs = plsc.load_gather(bufs3, [bv, r, hd])
            vd = plsc.load_gather(bufd3, [bv, r, hd + 4])
            al = vs + vd
            al = jnp.where(al >= 0.0, al, al * 0.2)
            plsc.store_scatter(exb3, [bv, r, col], jnp.exp(al))
            return cr

        lax.fori_loop(0, GB * ((CH * 8) // 16), lane, 0, unroll=4)
        sc = [pltpu.async_copy(exb3.at[b], den_sh.at[idxd2.at[b]], sem4, add=True)
              for b in range(GB)]
        for d in sc:
            d.wait()
        return carry

    lax.fori_loop(0, NG_D, group, 0)
    plsc.subcore_barrier()
    pltpu.sync_copy(den_sh.at[pl.ds(lo, NSTRIPE)],
                    den_h.at[c, pl.ds(lo, NSTRIPE)])


@functools.partial(
    pl.kernel,
    mesh=plsc.VectorSubcoreMesh(core_axis_name="c", subcore_axis_name="s"),
    compiler_params=pltpu.CompilerParams(
        use_tc_tiling_on_sc=False, needs_layout_passes=False),
    out_type=[jax.ShapeDtypeStruct((2, NP, 8), jnp.float32)],
    scratch_types=[
        pltpu.VMEM((GB, CH), jnp.int32),
        pltpu.VMEM((GB, CH), jnp.int32),
        pltpu.VMEM((GB, CH, 2 * HEADS), jnp.float32),
        pltpu.VMEM((GB, CH, 2 * HEADS), jnp.float32),
        pltpu.VMEM((GB, CH, 8), jnp.float32),
        pltpu.VMEM_SHARED((NP, 8), jnp.float32),
        pltpu.SemaphoreType.DMA,
        pltpu.SemaphoreType.DMA,
    ],
)
def _sc_den(src2_h, dst2_h, asd_h, z8_h, den_h,
            idxs2, idxd2, bufs3, bufd3, exb3, den_sh, sem1, sem4):
    _sc_den_body(src2_h, dst2_h, asd_h, z8_h, den_h,
                 idxs2, idxd2, bufs3, bufd3, exb3, den_sh, sem1, sem4)


def kernel(node_features, edge_index, edge_attr, pipeline_state,
           register_pressure, ready_mask, scheduled_mask,
           enc_W1, enc_b1, enc_W2, enc_b2,
           gat_W, gat_att_src, gat_att_dst, gat_bias,
           ln_g, ln_b,
           pip_W1, pip_b1, pip_W2, pip_b2):
    f32 = jnp.float32
    nf_pad = jnp.pad(node_features, ((0, NP - N), (0, 0)))

    loops = jnp.arange(N, dtype=edge_index.dtype)
    pad_e = jnp.full((EPAD - E - N,), DUMMY, dtype=edge_index.dtype)
    src = jnp.concatenate([edge_index[0], loops, pad_e]).reshape(NROW, CH)
    dst = jnp.concatenate([edge_index[1], loops, pad_e]).reshape(NROW, CH)

    z8 = jnp.zeros((NP, 8), f32)
    z32 = jnp.zeros((NP, 32), f32)
    sel = jnp.asarray(np.kron(np.eye(HEADS), np.ones((1, FH))), f32)
    eye = jnp.asarray(np.eye(HEADS), f32)

    b1r = enc_b1.reshape(1, HID)
    b2r = enc_b2.reshape(1, HID)
    h = _encode(nf_pad, enc_W1, b1r, enc_W2, b2r)

    for i in range(NLAYERS):
        a_src = (gat_att_src[i][:, :, None] * eye[:, None, :]).reshape(HID, HEADS)
        a_dst = (gat_att_dst[i][:, :, None] * eye[:, None, :]).reshape(HID, HEADS)
        acat = jnp.concatenate([a_src, a_dst], axis=1)
        x2, asd = _project(h, gat_W[i], acat)
        out2 = _sc_edge(src, dst, asd, x2, z32)[0]
        denp = _sc_den(src, dst, asd, z8)[0]
        h = _postprocess(h, out2, denp, sel,
                         gat_bias[i].reshape(1, HID),
                         ln_g[i].reshape(1, HID), ln_b[i].reshape(1, HID))

    pf = jnp.concatenate([pipeline_state, register_pressure])
    pf_pad = jnp.zeros((8, 16), f32).at[0, :9].set(pf)
    w1p = jnp.pad(pip_W1, ((0, 16 - 9), (0, 0)))
    q = _pipeline_mlp(pf_pad, w1p, pip_b1.reshape(1, HID),
                      pip_W2, pip_b2.reshape(1, HID))
    return (h[:N], q[0])


# speedup vs baseline: 70.3034x; 1.0403x over previous
import functools

import jax
import jax.numpy as jnp
import numpy as np
from jax import lax
from jax.experimental import pallas as pl
from jax.experimental.pallas import tpu as pltpu
from jax.experimental.pallas import tpu_sc as plsc

N = 50000
E = 800000
HID = 64
HEADS = 4
FH = 16
NFEAT = 48
NLAYERS = 3

NP = 50176
DUMMY = 50000
NB = 49
BR = 1024
CH = 128
EPAD = 851968
NSTRIPE = NP // 16


def _enc_body(nf, w1, b1, w2, b2, out):
    h1 = jnp.maximum(jnp.dot(nf[...], w1[...], preferred_element_type=jnp.float32)
                     + b1[...], 0.0)
    out[...] = jnp.dot(h1, w2[...], preferred_element_type=jnp.float32) + b2[...]


def _encode(nf_pad, w1, b1, w2, b2):
    return pl.pallas_call(
        _enc_body,
        grid=(NB,),
        in_specs=[
            pl.BlockSpec((BR, NFEAT), lambda i: (i, 0)),
            pl.BlockSpec((NFEAT, HID), lambda i: (0, 0)),
            pl.BlockSpec((1, HID), lambda i: (0, 0)),
            pl.BlockSpec((HID, HID), lambda i: (0, 0)),
            pl.BlockSpec((1, HID), lambda i: (0, 0)),
        ],
        out_specs=pl.BlockSpec((BR, HID), lambda i: (i, 0)),
        out_shape=jax.ShapeDtypeStruct((NP, HID), jnp.float32),
    )(nf_pad, w1, b1, w2, b2)


def _proj_body(h, w, a, x2, asd):
    x = jnp.dot(h[...], w[...], preferred_element_type=jnp.float32)
    asd[...] = jnp.dot(x, a[...], preferred_element_type=jnp.float32)
    j = pl.program_id(1)
    x2[...] = jnp.where(j == 0, x[:, :32], x[:, 32:])


def _project(h, w, acat):
    return pl.pallas_call(
        _proj_body,
        grid=(NB, 2),
        in_specs=[
            pl.BlockSpec((BR, HID), lambda i, j: (i, 0)),
            pl.BlockSpec((HID, HID), lambda i, j: (0, 0)),
            pl.BlockSpec((HID, 2 * HEADS), lambda i, j: (0, 0)),
        ],
        out_specs=[
            pl.BlockSpec((BR, 32), lambda i, j: (j * NB + i, 0)),
            pl.BlockSpec((BR, 2 * HEADS), lambda i, j: (i, 0)),
        ],
        out_shape=[
            jax.ShapeDtypeStruct((2 * NP, 32), jnp.float32),
            jax.ShapeDtypeStruct((NP, 2 * HEADS), jnp.float32),
        ],
    )(h, w, acat)


def _post_body(hres, olo, ohi, d0, d1, sel, bias, g, b, out):
    den = d0[...][0][:, :HEADS] + d1[...][0][:, :HEADS]
    inv = 1.0 / (den + 1e-16)
    inv64 = jnp.dot(inv, sel[...], preferred_element_type=jnp.float32)
    agg = jnp.concatenate([olo[...][0], ohi[...][0]], axis=-1)
    y = agg * inv64 + bias[...] + hres[...]
    m = jnp.mean(y, axis=-1, keepdims=True)
    yc = y - m
    var = jnp.mean(yc * yc, axis=-1, keepdims=True)
    out[...] = jnp.maximum(yc * lax.rsqrt(var + 1e-5) * g[...] + b[...], 0.0)


def _postprocess(hres, out2, denp, sel, bias, g, b):
    return pl.pallas_call(
        _post_body,
        grid=(NB,),
        in_specs=[
            pl.BlockSpec((BR, HID), lambda i: (i, 0)),
            pl.BlockSpec((1, BR, 32), lambda i: (0, i, 0)),
            pl.BlockSpec((1, BR, 32), lambda i: (1, i, 0)),
            pl.BlockSpec((1, BR, 8), lambda i: (0, i, 0)),
            pl.BlockSpec((1, BR, 8), lambda i: (1, i, 0)),
            pl.BlockSpec((HEADS, HID), lambda i: (0, 0)),
            pl.BlockSpec((1, HID), lambda i: (0, 0)),
            pl.BlockSpec((1, HID), lambda i: (0, 0)),
            pl.BlockSpec((1, HID), lambda i: (0, 0)),
        ],
        out_specs=pl.BlockSpec((BR, HID), lambda i: (i, 0)),
        out_shape=jax.ShapeDtypeStruct((NP, HID), jnp.float32),
    )(hres, out2, out2, denp, denp, sel, bias, g, b)


def _mlp_body(pf, w1, b1, w2, b2, out):
    h1 = jnp.maximum(jnp.dot(pf[...], w1[...], preferred_element_type=jnp.float32)
                     + b1[...], 0.0)
    out[...] = jnp.dot(h1, w2[...], preferred_element_type=jnp.float32) + b2[...]


def _pipeline_mlp(pf_pad, w1p, b1, w2, b2):
    return pl.pallas_call(
        _mlp_body,
        grid=(1,),
        in_specs=[
            pl.BlockSpec((8, 16), lambda i: (0, 0)),
            pl.BlockSpec((16, HID), lambda i: (0, 0)),
            pl.BlockSpec((1, HID), lambda i: (0, 0)),
            pl.BlockSpec((HID, HID), lambda i: (0, 0)),
            pl.BlockSpec((1, HID), lambda i: (0, 0)),
        ],
        out_specs=pl.BlockSpec((8, HID), lambda i: (0, 0)),
        out_shape=jax.ShapeDtypeStruct((8, HID), jnp.float32),
    )(pf_pad, w1p, b1, w2, b2)


GB = 4
NROW = EPAD // CH
NG_E = EPAD // 16 // (GB * CH)
GBD = 16
NG_D = EPAD // 32 // (GBD * CH)


def _sc_edge_body(src2_h, dst2_h, asd_h, x2_h, z32_h, out2_h,
                  idxs2, idxd2, bufs3, bufd3, xbuf3,
                  out_sh, sem1, sem3, sem4):
    c = lax.axis_index("c")
    s = lax.axis_index("s")
    lo = s * NSTRIPE
    pltpu.sync_copy(z32_h.at[pl.ds(lo, NSTRIPE)], out_sh.at[pl.ds(lo, NSTRIPE)])
    plsc.subcore_barrier()

    iota = lax.broadcasted_iota(jnp.int32, (16,), 0)
    rbase = s * (NROW // 16)
    xoff = c * NP
    cb = 2 * c

    def group(g, carry):
        r0 = rbase + g * GB
        pltpu.sync_copy(src2_h.at[pl.ds(r0, GB)], idxs2)
        pltpu.sync_copy(dst2_h.at[pl.ds(r0, GB)], idxd2)
        gs = [pltpu.async_copy(asd_h.at[idxs2.at[b]], bufs3.at[b], sem1)
              for b in range(GB)]
        gd = [pltpu.async_copy(asd_h.at[idxd2.at[b]], bufd3.at[b], sem1)
              for b in range(GB)]
        for d in gs:
            d.wait()
        for d in gd:
            d.wait()

        for b in range(GB):
            def adj(k, cr, b=b):
                sl = pl.ds(k * 16, 16)
                idxs2[b, sl] = idxs2[b, sl] + xoff
                return cr

            lax.fori_loop(0, CH // 16, adj, 0, unroll=8)
        gx = [pltpu.async_copy(x2_h.at[idxs2.at[b]], xbuf3.at[b], sem3)
              for b in range(GB)]

        for b in range(GB):
            bv = jnp.broadcast_to(b, (16,))

            def lane(j, cr, bv=bv):
                p = j * 16 + iota
                r = p // 4
                col = p % 4
                vs = plsc.load_gather(bufs3, [bv, r, col])
                vd = plsc.load_gather(bufd3, [bv, r, col + 4])
                al = vs + vd
                al = jnp.where(al >= 0.0, al, al * 0.2)
                plsc.store_scatter(bufd3, [bv, r, col], jnp.exp(al))
                return cr

            lax.fori_loop(0, (CH * 4) // 16, lane, 0, unroll=8)
        for d in gx:
            d.wait()

        cbv = jnp.broadcast_to(cb, (16,))
        cbv1 = jnp.broadcast_to(cb + 1, (16,))
        for b in range(GB):
            bv = jnp.broadcast_to(b, (16,))

            def edge(e, cr, bv=bv, b=b):
                ev = jnp.broadcast_to(e, (16,))
                c0 = plsc.load_gather(bufd3, [bv, ev, cbv])
                c1 = plsc.load_gather(bufd3, [bv, ev, cbv1])
                xbuf3[b, e, pl.ds(0, 16)] = xbuf3[b, e, pl.ds(0, 16)] * c0
                xbuf3[b, e, pl.ds(16, 16)] = xbuf3[b, e, pl.ds(16, 16)] * c1
                return cr

            lax.fori_loop(0, CH, edge, 0, unroll=8)
        sc = [pltpu.async_copy(xbuf3.at[b], out_sh.at[idxd2.at[b]], sem4, add=True)
              for b in range(GB)]
        for d in sc:
            d.wait()
        return carry

    lax.fori_loop(0, NG_E, group, 0)
    plsc.subcore_barrier()
    pltpu.sync_copy(out_sh.at[pl.ds(lo, NSTRIPE)],
                    out2_h.at[c, pl.ds(lo, NSTRIPE)])


@functools.partial(
    pl.kernel,
    mesh=plsc.VectorSubcoreMesh(core_axis_name="c", subcore_axis_name="s"),
    compiler_params=pltpu.CompilerParams(
        use_tc_tiling_on_sc=False, needs_layout_passes=False),
    out_type=[
        jax.ShapeDtypeStruct((2, NP, 32), jnp.float32),
    ],
    scratch_types=[
        pltpu.VMEM((GB, CH), jnp.int32),
        pltpu.VMEM((GB, CH), jnp.int32),
        pltpu.VMEM((GB, CH, 2 * HEADS), jnp.float32),
        pltpu.VMEM((GB, CH, 2 * HEADS), jnp.float32),
        pltpu.VMEM((GB, CH, 32), jnp.float32),
        pltpu.VMEM_SHARED((NP, 32), jnp.float32),
        pltpu.SemaphoreType.DMA,
        pltpu.SemaphoreType.DMA,
        pltpu.SemaphoreType.DMA,
    ],
)
def _sc_edge(src2_h, dst2_h, asd_h, x2_h, z32_h, out2_h,
             idxs2, idxd2, bufs3, bufd3, xbuf3,
             out_sh, sem1, sem3, sem4):
    _sc_edge_body(src2_h, dst2_h, asd_h, x2_h, z32_h, out2_h,
                  idxs2, idxd2, bufs3, bufd3, xbuf3,
                  out_sh, sem1, sem3, sem4)


def _sc_den_body(src2_h, dst2_h, asd_h, z8_h, den_h,
                 idxs2, idxd2, bufs3, bufd3, exb3, den_sh, sem1, sem4):
    c = lax.axis_index("c")
    s = lax.axis_index("s")
    lo = s * NSTRIPE
    pltpu.sync_copy(z8_h.at[pl.ds(lo, NSTRIPE)], den_sh.at[pl.ds(lo, NSTRIPE)])
    plsc.subcore_barrier()
    iota = lax.broadcasted_iota(jnp.int32, (16,), 0)
    rbase = (c * 16 + s) * (NROW // 32)

    def group(g, carry):
        r0 = rbase + g * GBD
        pltpu.sync_copy(src2_h.at[pl.ds(r0, GBD)], idxs2)
        pltpu.sync_copy(dst2_h.at[pl.ds(r0, GBD)], idxd2)
        gs = [pltpu.async_copy(asd_h.at[idxs2.at[b]], bufs3.at[b], sem1)
              for b in range(GBD)]
        gd = [pltpu.async_copy(asd_h.at[idxd2.at[b]], bufd3.at[b], sem1)
              for b in range(GBD)]
        for d in gs:
            d.wait()
        for d in gd:
            d.wait()

        for b in range(GBD):
            bv = jnp.broadcast_to(b, (16,))

            def lane(j, cr, bv=bv):
                p = j * 16 + iota
                r = p // 4
                col = p % 4
                vs = plsc.load_gather(bufs3, [bv, r, col])
                vd = plsc.load_gather(bufd3, [bv, r, col + 4])
                al = vs + vd
                al = jnp.where(al >= 0.0, al, al * 0.2)
                ev = jnp.exp(al)
                plsc.store_scatter(exb3, [bv, r, col], ev)
                plsc.store_scatter(exb3, [bv, r, col + 4], ev)
                return cr

            lax.fori_loop(0, (CH * 4) // 16, lane, 0, unroll=8)
        sc = [pltpu.async_copy(exb3.at[b], den_sh.at[idxd2.at[b]], sem4, add=True)
              for b in range(GBD)]
        for d in sc:
            d.wait()
        return carry

    lax.fori_loop(0, NG_D, group, 0)
    plsc.subcore_barrier()
    pltpu.sync_copy(den_sh.at[pl.ds(lo, NSTRIPE)],
                    den_h.at[c, pl.ds(lo, NSTRIPE)])


@functools.partial(
    pl.kernel,
    mesh=plsc.VectorSubcoreMesh(core_axis_name="c", subcore_axis_name="s"),
    compiler_params=pltpu.CompilerParams(
        use_tc_tiling_on_sc=False, needs_layout_passes=False),
    out_type=[jax.ShapeDtypeStruct((2, NP, 8), jnp.float32)],
    scratch_types=[
        pltpu.VMEM((GBD, CH), jnp.int32),
        pltpu.VMEM((GBD, CH), jnp.int32),
        pltpu.VMEM((GBD, CH, 2 * HEADS), jnp.float32),
        pltpu.VMEM((GBD, CH, 2 * HEADS), jnp.float32),
        pltpu.VMEM((GBD, CH, 8), jnp.float32),
        pltpu.VMEM_SHARED((NP, 8), jnp.float32),
        pltpu.SemaphoreType.DMA,
        pltpu.SemaphoreType.DMA,
    ],
)
def _sc_den(src2_h, dst2_h, asd_h, z8_h, den_h,
            idxs2, idxd2, bufs3, bufd3, exb3, den_sh, sem1, sem4):
    _sc_den_body(src2_h, dst2_h, asd_h, z8_h, den_h,
                 idxs2, idxd2, bufs3, bufd3, exb3, den_sh, sem1, sem4)


def kernel(node_features, edge_index, edge_attr, pipeline_state,
           register_pressure, ready_mask, scheduled_mask,
           enc_W1, enc_b1, enc_W2, enc_b2,
           gat_W, gat_att_src, gat_att_dst, gat_bias,
           ln_g, ln_b,
           pip_W1, pip_b1, pip_W2, pip_b2):
    f32 = jnp.float32
    nf_pad = jnp.pad(node_features, ((0, NP - N), (0, 0)))

    loops = jnp.arange(N, dtype=edge_index.dtype)
    pad_e = jnp.full((EPAD - E - N,), DUMMY, dtype=edge_index.dtype)
    src = jnp.concatenate([edge_index[0], loops, pad_e]).reshape(NROW, CH)
    dst = jnp.concatenate([edge_index[1], loops, pad_e]).reshape(NROW, CH)

    z8 = jnp.zeros((NP, 8), f32)
    z32 = jnp.zeros((NP, 32), f32)
    sel = jnp.asarray(np.kron(np.eye(HEADS), np.ones((1, FH))), f32)
    eye = jnp.asarray(np.eye(HEADS), f32)

    b1r = enc_b1.reshape(1, HID)
    b2r = enc_b2.reshape(1, HID)
    h = _encode(nf_pad, enc_W1, b1r, enc_W2, b2r)

    for i in range(NLAYERS):
        a_src = (gat_att_src[i][:, :, None] * eye[:, None, :]).reshape(HID, HEADS)
        a_dst = (gat_att_dst[i][:, :, None] * eye[:, None, :]).reshape(HID, HEADS)
        acat = jnp.concatenate([a_src, a_dst], axis=1)
        x2, asd = _project(h, gat_W[i], acat)
        out2 = _sc_edge(src, dst, asd, x2, z32)[0]
        denp = _sc_den(src, dst, asd, z8)[0]
        h = _postprocess(h, out2, denp, sel,
                         gat_bias[i].reshape(1, HID),
                         ln_g[i].reshape(1, HID), ln_b[i].reshape(1, HID))

    pf = jnp.concatenate([pipeline_state, register_pressure])
    pf_pad = jnp.zeros((8, 16), f32).at[0, :9].set(pf)
    w1p = jnp.pad(pip_W1, ((0, 16 - 9), (0, 0)))
    q = _pipeline_mlp(pf_pad, w1p, pip_b1.reshape(1, HID),
                      pip_W2, pip_b2.reshape(1, HID))
    return (h[:N], q[0])


# speedup vs baseline: 75.4179x; 1.0727x over previous
import functools

import jax
import jax.numpy as jnp
import numpy as np
from jax import lax
from jax.experimental import pallas as pl
from jax.experimental.pallas import tpu as pltpu
from jax.experimental.pallas import tpu_sc as plsc

N = 50000
E = 800000
HID = 64
HEADS = 4
FH = 16
NFEAT = 48
NLAYERS = 3

NP = 50176
DUMMY = 50000
NB = 49
BR = 1024
CH = 128
EPAD = 851968
NSTRIPE = NP // 16


def _enc_body(nf, w1, b1, w2, b2, out):
    h1 = jnp.maximum(jnp.dot(nf[...], w1[...], preferred_element_type=jnp.float32)
                     + b1[...], 0.0)
    out[...] = jnp.dot(h1, w2[...], preferred_element_type=jnp.float32) + b2[...]


def _encode(nf_pad, w1, b1, w2, b2):
    return pl.pallas_call(
        _enc_body,
        grid=(NB,),
        in_specs=[
            pl.BlockSpec((BR, NFEAT), lambda i: (i, 0)),
            pl.BlockSpec((NFEAT, HID), lambda i: (0, 0)),
            pl.BlockSpec((1, HID), lambda i: (0, 0)),
            pl.BlockSpec((HID, HID), lambda i: (0, 0)),
            pl.BlockSpec((1, HID), lambda i: (0, 0)),
        ],
        out_specs=pl.BlockSpec((BR, HID), lambda i: (i, 0)),
        out_shape=jax.ShapeDtypeStruct((NP, HID), jnp.float32),
    )(nf_pad, w1, b1, w2, b2)


def _proj_body(h, w, a, x2, asd):
    x = jnp.dot(h[...], w[...], preferred_element_type=jnp.float32)
    asd[...] = jnp.dot(x, a[...], preferred_element_type=jnp.float32)
    j = pl.program_id(1)
    x2[...] = jnp.where(j == 0, x[:, :32], x[:, 32:])


def _project(h, w, acat):
    return pl.pallas_call(
        _proj_body,
        grid=(NB, 2),
        in_specs=[
            pl.BlockSpec((BR, HID), lambda i, j: (i, 0)),
            pl.BlockSpec((HID, HID), lambda i, j: (0, 0)),
            pl.BlockSpec((HID, 2 * HEADS), lambda i, j: (0, 0)),
        ],
        out_specs=[
            pl.BlockSpec((BR, 32), lambda i, j: (j * NB + i, 0)),
            pl.BlockSpec((BR, 2 * HEADS), lambda i, j: (i, 0)),
        ],
        out_shape=[
            jax.ShapeDtypeStruct((2 * NP, 32), jnp.float32),
            jax.ShapeDtypeStruct((NP, 2 * HEADS), jnp.float32),
        ],
    )(h, w, acat)


def _post_body(hres, olo, ohi, d0, d1, sel, bias, g, b, out):
    den = d0[...][0][:, :HEADS] + d1[...][0][:, :HEADS]
    inv = 1.0 / (den + 1e-16)
    inv64 = jnp.dot(inv, sel[...], preferred_element_type=jnp.float32)
    agg = jnp.concatenate([olo[...][0], ohi[...][0]], axis=-1)
    y = agg * inv64 + bias[...] + hres[...]
    m = jnp.mean(y, axis=-1, keepdims=True)
    yc = y - m
    var = jnp.mean(yc * yc, axis=-1, keepdims=True)
    out[...] = jnp.maximum(yc * lax.rsqrt(var + 1e-5) * g[...] + b[...], 0.0)


def _postprocess(hres, out2, denp, sel, bias, g, b):
    return pl.pallas_call(
        _post_body,
        grid=(NB,),
        in_specs=[
            pl.BlockSpec((BR, HID), lambda i: (i, 0)),
            pl.BlockSpec((1, BR, 32), lambda i: (0, i, 0)),
            pl.BlockSpec((1, BR, 32), lambda i: (1, i, 0)),
            pl.BlockSpec((1, BR, 8), lambda i: (0, i, 0)),
            pl.BlockSpec((1, BR, 8), lambda i: (1, i, 0)),
            pl.BlockSpec((HEADS, HID), lambda i: (0, 0)),
            pl.BlockSpec((1, HID), lambda i: (0, 0)),
            pl.BlockSpec((1, HID), lambda i: (0, 0)),
            pl.BlockSpec((1, HID), lambda i: (0, 0)),
        ],
        out_specs=pl.BlockSpec((BR, HID), lambda i: (i, 0)),
        out_shape=jax.ShapeDtypeStruct((NP, HID), jnp.float32),
    )(hres, out2, out2, denp, denp, sel, bias, g, b)


def _mlp_body(pf, w1, b1, w2, b2, out):
    h1 = jnp.maximum(jnp.dot(pf[...], w1[...], preferred_element_type=jnp.float32)
                     + b1[...], 0.0)
    out[...] = jnp.dot(h1, w2[...], preferred_element_type=jnp.float32) + b2[...]


def _pipeline_mlp(pf_pad, w1p, b1, w2, b2):
    return pl.pallas_call(
        _mlp_body,
        grid=(1,),
        in_specs=[
            pl.BlockSpec((8, 16), lambda i: (0, 0)),
            pl.BlockSpec((16, HID), lambda i: (0, 0)),
            pl.BlockSpec((1, HID), lambda i: (0, 0)),
            pl.BlockSpec((HID, HID), lambda i: (0, 0)),
            pl.BlockSpec((1, HID), lambda i: (0, 0)),
        ],
        out_specs=pl.BlockSpec((8, HID), lambda i: (0, 0)),
        out_shape=jax.ShapeDtypeStruct((8, HID), jnp.float32),
    )(pf_pad, w1p, b1, w2, b2)


GB = 4
NROW = EPAD // CH
NG_E = EPAD // 16 // (GB * CH)
GBD = 16
NG_D = EPAD // 32 // (GBD * CH)


GBP = 2
NG_P = EPAD // 16 // (GBP * CH)


def _sc_edge_body(src2_h, dst2_h, asd_h, x2_h, z32_h, out2_h,
                  idxs2, idxx2, idxd2, bufs3, bufd3, xbuf3,
                  out_sh, sems0, sems1, semx0, semx1, semo0, semo1):
    c = lax.axis_index("c")
    s = lax.axis_index("s")
    lo = s * NSTRIPE
    pltpu.sync_copy(z32_h.at[pl.ds(lo, NSTRIPE)], out_sh.at[pl.ds(lo, NSTRIPE)])
    plsc.subcore_barrier()

    iota = lax.broadcasted_iota(jnp.int32, (16,), 0)
    rbase = s * (NROW // 16)
    xoff = c * NP
    cb = 2 * c
    cbv = jnp.broadcast_to(cb, (16,))
    cbv1 = jnp.broadcast_to(cb + 1, (16,))
    sems = (sems0, sems1)
    semx = (semx0, semx1)
    semo = (semo0, semo1)

    def issue(g, p):
        r0 = rbase + g * GBP
        pltpu.sync_copy(src2_h.at[pl.ds(r0, GBP)], idxs2.at[p])
        pltpu.sync_copy(dst2_h.at[pl.ds(r0, GBP)], idxd2.at[p])
        for b in range(GBP):
            pltpu.async_copy(asd_h.at[idxs2.at[p, b]], bufs3.at[p, b], sems[p])
            pltpu.async_copy(asd_h.at[idxd2.at[p, b]], bufd3.at[p, b], sems[p])
        for b in range(GBP):
            def adj(k, cr, p=p, b=b):
                sl = pl.ds(k * 16, 16)
                idxx2[p, b, sl] = idxs2[p, b, sl] + xoff
                return cr
            lax.fori_loop(0, CH // 16, adj, 0, unroll=8)
        for b in range(GBP):
            pltpu.async_copy(x2_h.at[idxx2.at[p, b]], xbuf3.at[p, b], semx[p])

    def drain_asd(p):
        for _ in range(2 * GBP):
            pltpu.make_async_copy(asd_h.at[pl.ds(0, CH)],
                                  bufs3.at[p, 0], sems[p]).wait()

    def drain_x(p):
        for _ in range(GBP):
            pltpu.make_async_copy(x2_h.at[pl.ds(0, CH)],
                                  xbuf3.at[p, 0], semx[p]).wait()

    def drain_out(p):
        for _ in range(GBP):
            pltpu.make_async_copy(x2_h.at[pl.ds(0, CH)],
                                  xbuf3.at[p, 0], semo[p]).wait()

    def compute(p, mid=None):
        drain_asd(p)
        pv = jnp.broadcast_to(p, (16,))
        for b in range(GBP):
            bv = jnp.broadcast_to(b, (16,))

            def lane(j, cr, bv=bv):
                pp = j * 16 + iota
                r = pp // 4
                col = pp % 4
                vs = plsc.load_gather(bufs3, [pv, bv, r, col])
                vd = plsc.load_gather(bufd3, [pv, bv, r, col + 4])
                al = vs + vd
                al = jnp.where(al >= 0.0, al, al * 0.2)
                plsc.store_scatter(bufd3, [pv, bv, r, col], jnp.exp(al))
                return cr

            lax.fori_loop(0, (CH * 4) // 16, lane, 0, unroll=8)
        if mid is not None:
            mid()
        drain_x(p)
        for b in range(GBP):
            bv = jnp.broadcast_to(b, (16,))

            def edge(e, cr, bv=bv, b=b):
                ev = jnp.broadcast_to(e, (16,))
                c0 = plsc.load_gather(bufd3, [pv, bv, ev, cbv])
                c1 = plsc.load_gather(bufd3, [pv, bv, ev, cbv1])
                xbuf3[p, b, e, pl.ds(0, 16)] = xbuf3[p, b, e, pl.ds(0, 16)] * c0
                xbuf3[p, b, e, pl.ds(16, 16)] = xbuf3[p, b, e, pl.ds(16, 16)] * c1
                return cr

            lax.fori_loop(0, CH, edge, 0, unroll=8)
        for b in range(GBP):
            pltpu.async_copy(xbuf3.at[p, b], out_sh.at[idxd2.at[p, b]],
                             semo[p], add=True)

    issue(0, 0)
    NPAIR = NG_P // 2

    def pair(g2, carry):
        g = 2 * g2

        @pl.when(g2 > 0)
        def _():
            drain_out(1)
        issue(g + 1, 1)

        def mid0():
            pass

        compute(0, mid0)

        def mid1():
            @pl.when(g2 < NPAIR - 1)
            def _():
                drain_out(0)
                issue(g + 2, 0)

        compute(1, mid1)
        return carry

    lax.fori_loop(0, NPAIR, pair, 0)
    drain_out(0)
    drain_out(1)
    plsc.subcore_barrier()
    pltpu.sync_copy(out_sh.at[pl.ds(lo, NSTRIPE)],
                    out2_h.at[c, pl.ds(lo, NSTRIPE)])


@functools.partial(
    pl.kernel,
    mesh=plsc.VectorSubcoreMesh(core_axis_name="c", subcore_axis_name="s"),
    compiler_params=pltpu.CompilerParams(
        use_tc_tiling_on_sc=False, needs_layout_passes=False),
    out_type=[
        jax.ShapeDtypeStruct((2, NP, 32), jnp.float32),
    ],
    scratch_types=[
        pltpu.VMEM((2, GBP, CH), jnp.int32),
        pltpu.VMEM((2, GBP, CH), jnp.int32),
        pltpu.VMEM((2, GBP, CH), jnp.int32),
        pltpu.VMEM((2, GBP, CH, 2 * HEADS), jnp.float32),
        pltpu.VMEM((2, GBP, CH, 2 * HEADS), jnp.float32),
        pltpu.VMEM((2, GBP, CH, 32), jnp.float32),
        pltpu.VMEM_SHARED((NP, 32), jnp.float32),
        pltpu.SemaphoreType.DMA,
        pltpu.SemaphoreType.DMA,
        pltpu.SemaphoreType.DMA,
        pltpu.SemaphoreType.DMA,
        pltpu.SemaphoreType.DMA,
        pltpu.SemaphoreType.DMA,
    ],
)
def _sc_edge(src2_h, dst2_h, asd_h, x2_h, z32_h, out2_h,
             idxs2, idxx2, idxd2, bufs3, bufd3, xbuf3,
             out_sh, sems0, sems1, semx0, semx1, semo0, semo1):
    _sc_edge_body(src2_h, dst2_h, asd_h, x2_h, z32_h, out2_h,
                  idxs2, idxx2, idxd2, bufs3, bufd3, xbuf3,
                  out_sh, sems0, sems1, semx0, semx1, semo0, semo1)


def _sc_den_body(src2_h, dst2_h, asd_h, z8_h, den_h,
                 idxs2, idxd2, bufs3, bufd3, exb3, den_sh, sem1, sem4):
    c = lax.axis_index("c")
    s = lax.axis_index("s")
    lo = s * NSTRIPE
    pltpu.sync_copy(z8_h.at[pl.ds(lo, NSTRIPE)], den_sh.at[pl.ds(lo, NSTRIPE)])
    plsc.subcore_barrier()
    iota = lax.broadcasted_iota(jnp.int32, (16,), 0)
    rbase = (c * 16 + s) * (NROW // 32)

    def group(g, carry):
        r0 = rbase + g * GBD
        pltpu.sync_copy(src2_h.at[pl.ds(r0, GBD)], idxs2)
        pltpu.sync_copy(dst2_h.at[pl.ds(r0, GBD)], idxd2)
        gs = [pltpu.async_copy(asd_h.at[idxs2.at[b]], bufs3.at[b], sem1)
              for b in range(GBD)]
        gd = [pltpu.async_copy(asd_h.at[idxd2.at[b]], bufd3.at[b], sem1)
              for b in range(GBD)]
        for d in gs:
            d.wait()
        for d in gd:
            d.wait()

        for b in range(GBD):
            bv = jnp.broadcast_to(b, (16,))

            def lane(j, cr, bv=bv):
                p = j * 16 + iota
                r = p // 4
                col = p % 4
                vs = plsc.load_gather(bufs3, [bv, r, col])
                vd = plsc.load_gather(bufd3, [bv, r, col + 4])
                al = vs + vd
                al = jnp.where(al >= 0.0, al, al * 0.2)
                ev = jnp.exp(al)
                plsc.store_scatter(exb3, [bv, r, col], ev)
                plsc.store_scatter(exb3, [bv, r, col + 4], ev)
                return cr

            lax.fori_loop(0, (CH * 4) // 16, lane, 0, unroll=8)
        sc = [pltpu.async_copy(exb3.at[b], den_sh.at[idxd2.at[b]], sem4, add=True)
              for b in range(GBD)]
        for d in sc:
            d.wait()
        return carry

    lax.fori_loop(0, NG_D, group, 0)
    plsc.subcore_barrier()
    pltpu.sync_copy(den_sh.at[pl.ds(lo, NSTRIPE)],
                    den_h.at[c, pl.ds(lo, NSTRIPE)])


@functools.partial(
    pl.kernel,
    mesh=plsc.VectorSubcoreMesh(core_axis_name="c", subcore_axis_name="s"),
    compiler_params=pltpu.CompilerParams(
        use_tc_tiling_on_sc=False, needs_layout_passes=False),
    out_type=[jax.ShapeDtypeStruct((2, NP, 8), jnp.float32)],
    scratch_types=[
        pltpu.VMEM((GBD, CH), jnp.int32),
        pltpu.VMEM((GBD, CH), jnp.int32),
        pltpu.VMEM((GBD, CH, 2 * HEADS), jnp.float32),
        pltpu.VMEM((GBD, CH, 2 * HEADS), jnp.float32),
        pltpu.VMEM((GBD, CH, 8), jnp.float32),
        pltpu.VMEM_SHARED((NP, 8), jnp.float32),
        pltpu.SemaphoreType.DMA,
        pltpu.SemaphoreType.DMA,
    ],
)
def _sc_den(src2_h, dst2_h, asd_h, z8_h, den_h,
            idxs2, idxd2, bufs3, bufd3, exb3, den_sh, sem1, sem4):
    _sc_den_body(src2_h, dst2_h, asd_h, z8_h, den_h,
                 idxs2, idxd2, bufs3, bufd3, exb3, den_sh, sem1, sem4)


def kernel(node_features, edge_index, edge_attr, pipeline_state,
           register_pressure, ready_mask, scheduled_mask,
           enc_W1, enc_b1, enc_W2, enc_b2,
           gat_W, gat_att_src, gat_att_dst, gat_bias,
           ln_g, ln_b,
           pip_W1, pip_b1, pip_W2, pip_b2):
    f32 = jnp.float32
    nf_pad = jnp.pad(node_features, ((0, NP - N), (0, 0)))

    loops = jnp.arange(N, dtype=edge_index.dtype)
    pad_e = jnp.full((EPAD - E - N,), DUMMY, dtype=edge_index.dtype)
    src = jnp.concatenate([edge_index[0], loops, pad_e]).reshape(NROW, CH)
    dst = jnp.concatenate([edge_index[1], loops, pad_e]).reshape(NROW, CH)

    z8 = jnp.zeros((NP, 8), f32)
    z32 = jnp.zeros((NP, 32), f32)
    sel = jnp.asarray(np.kron(np.eye(HEADS), np.ones((1, FH))), f32)
    eye = jnp.asarray(np.eye(HEADS), f32)

    b1r = enc_b1.reshape(1, HID)
    b2r = enc_b2.reshape(1, HID)
    h = _encode(nf_pad, enc_W1, b1r, enc_W2, b2r)

    for i in range(NLAYERS):
        a_src = (gat_att_src[i][:, :, None] * eye[:, None, :]).reshape(HID, HEADS)
        a_dst = (gat_att_dst[i][:, :, None] * eye[:, None, :]).reshape(HID, HEADS)
        acat = jnp.concatenate([a_src, a_dst], axis=1)
        x2, asd = _project(h, gat_W[i], acat)
        out2 = _sc_edge(src, dst, asd, x2, z32)[0]
        denp = _sc_den(src, dst, asd, z8)[0]
        h = _postprocess(h, out2, denp, sel,
                         gat_bias[i].reshape(1, HID),
                         ln_g[i].reshape(1, HID), ln_b[i].reshape(1, HID))

    pf = jnp.concatenate([pipeline_state, register_pressure])
    pf_pad = jnp.zeros((8, 16), f32).at[0, :9].set(pf)
    w1p = jnp.pad(pip_W1, ((0, 16 - 9), (0, 0)))
    q = _pipeline_mlp(pf_pad, w1p, pip_b1.reshape(1, HID),
                      pip_W2, pip_b2.reshape(1, HID))
    return (h[:N], q[0])


# speedup vs baseline: 77.2292x; 1.0240x over previous
import functools

import jax
import jax.numpy as jnp
import numpy as np
from jax import lax
from jax.experimental import pallas as pl
from jax.experimental.pallas import tpu as pltpu
from jax.experimental.pallas import tpu_sc as plsc

N = 50000
E = 800000
HID = 64
HEADS = 4
FH = 16
NFEAT = 48
NLAYERS = 3

NP = 50176
DUMMY = 50000
NB = 49
BR = 1024
CH = 128
EPAD = 851968
NSTRIPE = NP // 16


def _enc_body(nf, w1, b1, w2, b2, out):
    h1 = jnp.maximum(jnp.dot(nf[...], w1[...], preferred_element_type=jnp.float32)
                     + b1[...], 0.0)
    out[...] = jnp.dot(h1, w2[...], preferred_element_type=jnp.float32) + b2[...]


def _encode(nf_pad, w1, b1, w2, b2):
    return pl.pallas_call(
        _enc_body,
        grid=(NB,),
        in_specs=[
            pl.BlockSpec((BR, NFEAT), lambda i: (i, 0)),
            pl.BlockSpec((NFEAT, HID), lambda i: (0, 0)),
            pl.BlockSpec((1, HID), lambda i: (0, 0)),
            pl.BlockSpec((HID, HID), lambda i: (0, 0)),
            pl.BlockSpec((1, HID), lambda i: (0, 0)),
        ],
        out_specs=pl.BlockSpec((BR, HID), lambda i: (i, 0)),
        out_shape=jax.ShapeDtypeStruct((NP, HID), jnp.float32),
    )(nf_pad, w1, b1, w2, b2)


def _proj_body(h, w, a, x2, asd):
    x = jnp.dot(h[...], w[...], preferred_element_type=jnp.float32)
    asd[...] = jnp.dot(x, a[...], preferred_element_type=jnp.float32)
    j = pl.program_id(1)
    x2[...] = jnp.where(j == 0, x[:, :32], x[:, 32:])


def _project(h, w, acat):
    return pl.pallas_call(
        _proj_body,
        grid=(NB, 2),
        in_specs=[
            pl.BlockSpec((BR, HID), lambda i, j: (i, 0)),
            pl.BlockSpec((HID, HID), lambda i, j: (0, 0)),
            pl.BlockSpec((HID, 2 * HEADS), lambda i, j: (0, 0)),
        ],
        out_specs=[
            pl.BlockSpec((BR, 32), lambda i, j: (j * NB + i, 0)),
            pl.BlockSpec((BR, 2 * HEADS), lambda i, j: (i, 0)),
        ],
        out_shape=[
            jax.ShapeDtypeStruct((2 * NP, 32), jnp.float32),
            jax.ShapeDtypeStruct((NP, 2 * HEADS), jnp.float32),
        ],
    )(h, w, acat)


def _post_body(hres, olo, ohi, d0, d1, sel, bias, g, b, out):
    den = d0[...][0][:, :HEADS] + d1[...][0][:, :HEADS]
    inv = 1.0 / (den + 1e-16)
    inv64 = jnp.dot(inv, sel[...], preferred_element_type=jnp.float32)
    agg = jnp.concatenate([olo[...][0], ohi[...][0]], axis=-1)
    y = agg * inv64 + bias[...] + hres[...]
    m = jnp.mean(y, axis=-1, keepdims=True)
    yc = y - m
    var = jnp.mean(yc * yc, axis=-1, keepdims=True)
    out[...] = jnp.maximum(yc * lax.rsqrt(var + 1e-5) * g[...] + b[...], 0.0)


def _postprocess(hres, out2, denp, sel, bias, g, b):
    return pl.pallas_call(
        _post_body,
        grid=(NB,),
        in_specs=[
            pl.BlockSpec((BR, HID), lambda i: (i, 0)),
            pl.BlockSpec((1, BR, 32), lambda i: (0, i, 0)),
            pl.BlockSpec((1, BR, 32), lambda i: (1, i, 0)),
            pl.BlockSpec((1, BR, 8), lambda i: (0, i, 0)),
            pl.BlockSpec((1, BR, 8), lambda i: (1, i, 0)),
            pl.BlockSpec((HEADS, HID), lambda i: (0, 0)),
            pl.BlockSpec((1, HID), lambda i: (0, 0)),
            pl.BlockSpec((1, HID), lambda i: (0, 0)),
            pl.BlockSpec((1, HID), lambda i: (0, 0)),
        ],
        out_specs=pl.BlockSpec((BR, HID), lambda i: (i, 0)),
        out_shape=jax.ShapeDtypeStruct((NP, HID), jnp.float32),
    )(hres, out2, out2, denp, denp, sel, bias, g, b)


def _mlp_body(pf, w1, b1, w2, b2, out):
    h1 = jnp.maximum(jnp.dot(pf[...], w1[...], preferred_element_type=jnp.float32)
                     + b1[...], 0.0)
    out[...] = jnp.dot(h1, w2[...], preferred_element_type=jnp.float32) + b2[...]


def _pipeline_mlp(pf_pad, w1p, b1, w2, b2):
    return pl.pallas_call(
        _mlp_body,
        grid=(1,),
        in_specs=[
            pl.BlockSpec((8, 16), lambda i: (0, 0)),
            pl.BlockSpec((16, HID), lambda i: (0, 0)),
            pl.BlockSpec((1, HID), lambda i: (0, 0)),
            pl.BlockSpec((HID, HID), lambda i: (0, 0)),
            pl.BlockSpec((1, HID), lambda i: (0, 0)),
        ],
        out_specs=pl.BlockSpec((8, HID), lambda i: (0, 0)),
        out_shape=jax.ShapeDtypeStruct((8, HID), jnp.float32),
    )(pf_pad, w1p, b1, w2, b2)


GB = 4
NROW = EPAD // CH
NG_E = EPAD // 16 // (GB * CH)
GBD = 16
NG_D = EPAD // 32 // (GBD * CH)


GBP = 2
NG_P = EPAD // 16 // (GBP * CH)


def _sc_edge_body(src2_h, dst2_h, asd_h, x2_h, z32_h, out2_h,
                  idxs2, idxx2, idxd2, bufs3, bufd3, xbuf3,
                  out_sh, sems0, sems1, semx0, semx1, semo0, semo1):
    c = lax.axis_index("c")
    s = lax.axis_index("s")
    lo = s * NSTRIPE
    pltpu.sync_copy(z32_h.at[pl.ds(lo, NSTRIPE)], out_sh.at[pl.ds(lo, NSTRIPE)])
    plsc.subcore_barrier()

    iota = lax.broadcasted_iota(jnp.int32, (16,), 0)
    rbase = s * (NROW // 16)
    xoff = c * NP
    cb = 2 * c
    cbv = jnp.broadcast_to(cb, (16,))
    cbv1 = jnp.broadcast_to(cb + 1, (16,))
    sems = (sems0, sems1)
    semx = (semx0, semx1)
    semo = (semo0, semo1)

    def issue(g, p):
        r0 = rbase + g * GBP
        pltpu.sync_copy(src2_h.at[pl.ds(r0, GBP)], idxs2.at[p])
        pltpu.sync_copy(dst2_h.at[pl.ds(r0, GBP)], idxd2.at[p])
        for b in range(GBP):
            pltpu.async_copy(asd_h.at[idxs2.at[p, b]], bufs3.at[p, b], sems[p])
            pltpu.async_copy(asd_h.at[idxd2.at[p, b]], bufd3.at[p, b], sems[p])
        for b in range(GBP):
            def adj(k, cr, p=p, b=b):
                sl = pl.ds(k * 16, 16)
                idxx2[p, b, sl] = idxs2[p, b, sl] + xoff
                return cr
            lax.fori_loop(0, CH // 16, adj, 0, unroll=8)
        for b in range(GBP):
            pltpu.async_copy(x2_h.at[idxx2.at[p, b]], xbuf3.at[p, b], semx[p])

    def drain_asd(p):
        for _ in range(2 * GBP):
            pltpu.make_async_copy(asd_h.at[pl.ds(0, CH)],
                                  bufs3.at[p, 0], sems[p]).wait()

    def drain_x(p):
        for _ in range(GBP):
            pltpu.make_async_copy(x2_h.at[pl.ds(0, CH)],
                                  xbuf3.at[p, 0], semx[p]).wait()

    def drain_out(p):
        for _ in range(GBP):
            pltpu.make_async_copy(x2_h.at[pl.ds(0, CH)],
                                  xbuf3.at[p, 0], semo[p]).wait()

    def compute(p, mid=None):
        drain_asd(p)
        pv = jnp.broadcast_to(p, (16,))
        for b in range(GBP):
            bv = jnp.broadcast_to(b, (16,))

            def lane(j, cr, bv=bv):
                pp = j * 16 + iota
                r = pp // 4
                col = pp % 4
                vs = plsc.load_gather(bufs3, [pv, bv, r, col])
                vd = plsc.load_gather(bufd3, [pv, bv, r, col + 4])
                al = vs + vd
                al = jnp.where(al >= 0.0, al, al * 0.2)
                plsc.store_scatter(bufd3, [pv, bv, r, col], jnp.exp(al))
                return cr

            lax.fori_loop(0, (CH * 4) // 16, lane, 0, unroll=8)
        if mid is not None:
            mid()
        drain_x(p)
        for b in range(GBP):
            bv = jnp.broadcast_to(b, (16,))

            def edge(e, cr, bv=bv, b=b):
                ev = jnp.broadcast_to(e, (16,))
                c0 = plsc.load_gather(bufd3, [pv, bv, ev, cbv])
                c1 = plsc.load_gather(bufd3, [pv, bv, ev, cbv1])
                xbuf3[p, b, e, pl.ds(0, 16)] = xbuf3[p, b, e, pl.ds(0, 16)] * c0
                xbuf3[p, b, e, pl.ds(16, 16)] = xbuf3[p, b, e, pl.ds(16, 16)] * c1
                return cr

            lax.fori_loop(0, CH, edge, 0, unroll=8)
        for b in range(GBP):
            pltpu.async_copy(xbuf3.at[p, b], out_sh.at[idxd2.at[p, b]],
                             semo[p], add=True)

    issue(0, 0)
    NPAIR = NG_P // 2

    def pair(g2, carry):
        g = 2 * g2

        @pl.when(g2 > 0)
        def _():
            drain_out(1)
        issue(g + 1, 1)

        def mid0():
            pass

        compute(0, mid0)

        def mid1():
            @pl.when(g2 < NPAIR - 1)
            def _():
                drain_out(0)
                issue(g + 2, 0)

        compute(1, mid1)
        return carry

    lax.fori_loop(0, NPAIR, pair, 0)
    drain_out(0)
    drain_out(1)
    plsc.subcore_barrier()
    pltpu.sync_copy(out_sh.at[pl.ds(lo, NSTRIPE)],
                    out2_h.at[c, pl.ds(lo, NSTRIPE)])


@functools.partial(
    pl.kernel,
    mesh=plsc.VectorSubcoreMesh(core_axis_name="c", subcore_axis_name="s"),
    compiler_params=pltpu.CompilerParams(
        use_tc_tiling_on_sc=False, needs_layout_passes=False),
    out_type=[
        jax.ShapeDtypeStruct((2, NP, 32), jnp.float32),
    ],
    scratch_types=[
        pltpu.VMEM((2, GBP, CH), jnp.int32),
        pltpu.VMEM((2, GBP, CH), jnp.int32),
        pltpu.VMEM((2, GBP, CH), jnp.int32),
        pltpu.VMEM((2, GBP, CH, 2 * HEADS), jnp.float32),
        pltpu.VMEM((2, GBP, CH, 2 * HEADS), jnp.float32),
        pltpu.VMEM((2, GBP, CH, 32), jnp.float32),
        pltpu.VMEM_SHARED((NP, 32), jnp.float32),
        pltpu.SemaphoreType.DMA,
        pltpu.SemaphoreType.DMA,
        pltpu.SemaphoreType.DMA,
        pltpu.SemaphoreType.DMA,
        pltpu.SemaphoreType.DMA,
        pltpu.SemaphoreType.DMA,
    ],
)
def _sc_edge(src2_h, dst2_h, asd_h, x2_h, z32_h, out2_h,
             idxs2, idxx2, idxd2, bufs3, bufd3, xbuf3,
             out_sh, sems0, sems1, semx0, semx1, semo0, semo1):
    _sc_edge_body(src2_h, dst2_h, asd_h, x2_h, z32_h, out2_h,
                  idxs2, idxx2, idxd2, bufs3, bufd3, xbuf3,
                  out_sh, sems0, sems1, semx0, semx1, semo0, semo1)


GBD = 8
NG_D = EPAD // 32 // (GBD * CH)


def _sc_den_body(src2_h, dst2_h, asd_h, z8_h, den_h,
                 idxs2, idxd2, bufs3, bufd3, exb3,
                 den_sh, sems0, sems1, semo0, semo1):
    c = lax.axis_index("c")
    s = lax.axis_index("s")
    lo = s * NSTRIPE
    pltpu.sync_copy(z8_h.at[pl.ds(lo, NSTRIPE)], den_sh.at[pl.ds(lo, NSTRIPE)])
    plsc.subcore_barrier()
    iota = lax.broadcasted_iota(jnp.int32, (16,), 0)
    rbase = (c * 16 + s) * (NROW // 32)
    sems = (sems0, sems1)
    semo = (semo0, semo1)

    def issue(g, p):
        r0 = rbase + g * GBD
        pltpu.sync_copy(src2_h.at[pl.ds(r0, GBD)], idxs2.at[p])
        pltpu.sync_copy(dst2_h.at[pl.ds(r0, GBD)], idxd2.at[p])
        for b in range(GBD):
            pltpu.async_copy(asd_h.at[idxs2.at[p, b]], bufs3.at[p, b], sems[p])
            pltpu.async_copy(asd_h.at[idxd2.at[p, b]], bufd3.at[p, b], sems[p])

    def drain_asd(p):
        for _ in range(2 * GBD):
            pltpu.make_async_copy(asd_h.at[pl.ds(0, CH)],
                                  bufs3.at[p, 0], sems[p]).wait()

    def drain_out(p):
        for _ in range(GBD):
            pltpu.make_async_copy(asd_h.at[pl.ds(0, CH)],
                                  exb3.at[p, 0], semo[p]).wait()

    def compute(p, mid=None):
        drain_asd(p)
        pv = jnp.broadcast_to(p, (16,))
        for b in range(GBD):
            bv = jnp.broadcast_to(b, (16,))

            def lane(j, cr, bv=bv):
                pp = j * 16 + iota
                r = pp // 4
                col = pp % 4
                vs = plsc.load_gather(bufs3, [pv, bv, r, col])
                vd = plsc.load_gather(bufd3, [pv, bv, r, col + 4])
                al = vs + vd
                al = jnp.where(al >= 0.0, al, al * 0.2)
                ev = jnp.exp(al)
                plsc.store_scatter(exb3, [pv, bv, r, col], ev)
                plsc.store_scatter(exb3, [pv, bv, r, col + 4], ev)
                return cr

            lax.fori_loop(0, (CH * 4) // 16, lane, 0, unroll=8)
            if b == 0 and mid is not None:
                mid()
        for b in range(GBD):
            pltpu.async_copy(exb3.at[p, b], den_sh.at[idxd2.at[p, b]],
                             semo[p], add=True)

    issue(0, 0)
    NPAIR = NG_D // 2

    def pair(g2, carry):
        g = 2 * g2

        @pl.when(g2 > 0)
        def _():
            drain_out(1)
        issue(g + 1, 1)
        compute(0)

        def mid1():
            @pl.when(g2 < NPAIR - 1)
            def _():
                drain_out(0)
                issue(g + 2, 0)

        compute(1, mid1)
        return carry

    lax.fori_loop(0, NPAIR, pair, 0)
    drain_out(0)
    drain_out(1)
    plsc.subcore_barrier()
    pltpu.sync_copy(den_sh.at[pl.ds(lo, NSTRIPE)],
                    den_h.at[c, pl.ds(lo, NSTRIPE)])


@functools.partial(
    pl.kernel,
    mesh=plsc.VectorSubcoreMesh(core_axis_name="c", subcore_axis_name="s"),
    compiler_params=pltpu.CompilerParams(
        use_tc_tiling_on_sc=False, needs_layout_passes=False),
    out_type=[jax.ShapeDtypeStruct((2, NP, 8), jnp.float32)],
    scratch_types=[
        pltpu.VMEM((2, GBD, CH), jnp.int32),
        pltpu.VMEM((2, GBD, CH), jnp.int32),
        pltpu.VMEM((2, GBD, CH, 2 * HEADS), jnp.float32),
        pltpu.VMEM((2, GBD, CH, 2 * HEADS), jnp.float32),
        pltpu.VMEM((2, GBD, CH, 8), jnp.float32),
        pltpu.VMEM_SHARED((NP, 8), jnp.float32),
        pltpu.SemaphoreType.DMA,
        pltpu.SemaphoreType.DMA,
        pltpu.SemaphoreType.DMA,
        pltpu.SemaphoreType.DMA,
    ],
)
def _sc_den(src2_h, dst2_h, asd_h, z8_h, den_h,
            idxs2, idxd2, bufs3, bufd3, exb3,
            den_sh, sems0, sems1, semo0, semo1):
    _sc_den_body(src2_h, dst2_h, asd_h, z8_h, den_h,
                 idxs2, idxd2, bufs3, bufd3, exb3,
                 den_sh, sems0, sems1, semo0, semo1)


def kernel(node_features, edge_index, edge_attr, pipeline_state,
           register_pressure, ready_mask, scheduled_mask,
           enc_W1, enc_b1, enc_W2, enc_b2,
           gat_W, gat_att_src, gat_att_dst, gat_bias,
           ln_g, ln_b,
           pip_W1, pip_b1, pip_W2, pip_b2):
    f32 = jnp.float32
    nf_pad = jnp.pad(node_features, ((0, NP - N), (0, 0)))

    loops = jnp.arange(N, dtype=edge_index.dtype)
    pad_e = jnp.full((EPAD - E - N,), DUMMY, dtype=edge_index.dtype)
    src = jnp.concatenate([edge_index[0], loops, pad_e]).reshape(NROW, CH)
    dst = jnp.concatenate([edge_index[1], loops, pad_e]).reshape(NROW, CH)

    z8 = jnp.zeros((NP, 8), f32)
    z32 = jnp.zeros((NP, 32), f32)
    sel = jnp.asarray(np.kron(np.eye(HEADS), np.ones((1, FH))), f32)
    eye = jnp.asarray(np.eye(HEADS), f32)

    b1r = enc_b1.reshape(1, HID)
    b2r = enc_b2.reshape(1, HID)
    h = _encode(nf_pad, enc_W1, b1r, enc_W2, b2r)

    for i in range(NLAYERS):
        a_src = (gat_att_src[i][:, :, None] * eye[:, None, :]).reshape(HID, HEADS)
        a_dst = (gat_att_dst[i][:, :, None] * eye[:, None, :]).reshape(HID, HEADS)
        acat = jnp.concatenate([a_src, a_dst], axis=1)
        x2, asd = _project(h, gat_W[i], acat)
        out2 = _sc_edge(src, dst, asd, x2, z32)[0]
        denp = _sc_den(src, dst, asd, z8)[0]
        h = _postprocess(h, out2, denp, sel,
                         gat_bias[i].reshape(1, HID),
                         ln_g[i].reshape(1, HID), ln_b[i].reshape(1, HID))

    pf = jnp.concatenate([pipeline_state, register_pressure])
    pf_pad = jnp.zeros((8, 16), f32).at[0, :9].set(pf)
    w1p = jnp.pad(pip_W1, ((0, 16 - 9), (0, 0)))
    q = _pipeline_mlp(pf_pad, w1p, pip_b1.reshape(1, HID),
                      pip_W2, pip_b2.reshape(1, HID))
    return (h[:N], q[0])


# speedup vs baseline: 77.4029x; 1.0022x over previous
import functools

import jax
import jax.numpy as jnp
import numpy as np
from jax import lax
from jax.experimental import pallas as pl
from jax.experimental.pallas import tpu as pltpu
from jax.experimental.pallas import tpu_sc as plsc

N = 50000
E = 800000
HID = 64
HEADS = 4
FH = 16
NFEAT = 48
NLAYERS = 3

NP = 50176
DUMMY = 50000
NB = 49
BR = 1024
CH = 128
EPAD = 851968
NSTRIPE = NP // 16


def _enc_body(nf, w1, b1, w2, b2, out):
    h1 = jnp.maximum(jnp.dot(nf[...], w1[...], preferred_element_type=jnp.float32)
                     + b1[...], 0.0)
    out[...] = jnp.dot(h1, w2[...], preferred_element_type=jnp.float32) + b2[...]


def _encode(nf_pad, w1, b1, w2, b2):
    return pl.pallas_call(
        _enc_body,
        grid=(NB,),
        in_specs=[
            pl.BlockSpec((BR, NFEAT), lambda i: (i, 0)),
            pl.BlockSpec((NFEAT, HID), lambda i: (0, 0)),
            pl.BlockSpec((1, HID), lambda i: (0, 0)),
            pl.BlockSpec((HID, HID), lambda i: (0, 0)),
            pl.BlockSpec((1, HID), lambda i: (0, 0)),
        ],
        out_specs=pl.BlockSpec((BR, HID), lambda i: (i, 0)),
        out_shape=jax.ShapeDtypeStruct((NP, HID), jnp.float32),
    )(nf_pad, w1, b1, w2, b2)


def _proj_body(h, w, a, x2, asd):
    x = jnp.dot(h[...], w[...], preferred_element_type=jnp.float32)
    asd[...] = jnp.dot(x, a[...], preferred_element_type=jnp.float32)
    j = pl.program_id(1)
    x2[...] = jnp.where(j == 0, x[:, :32], x[:, 32:])


def _project(h, w, acat):
    return pl.pallas_call(
        _proj_body,
        grid=(NB, 2),
        in_specs=[
            pl.BlockSpec((BR, HID), lambda i, j: (i, 0)),
            pl.BlockSpec((HID, HID), lambda i, j: (0, 0)),
            pl.BlockSpec((HID, 2 * HEADS), lambda i, j: (0, 0)),
        ],
        out_specs=[
            pl.BlockSpec((BR, 32), lambda i, j: (j * NB + i, 0)),
            pl.BlockSpec((BR, 2 * HEADS), lambda i, j: (i, 0)),
        ],
        out_shape=[
            jax.ShapeDtypeStruct((2 * NP, 32), jnp.float32),
            jax.ShapeDtypeStruct((NP, 2 * HEADS), jnp.float32),
        ],
    )(h, w, acat)


def _encproj_body(nf, w1, b1, w2, b2, w, a, h_o, x2_o, asd_o):
    h1 = jnp.maximum(jnp.dot(nf[...], w1[...], preferred_element_type=jnp.float32)
                     + b1[...], 0.0)
    h = jnp.dot(h1, w2[...], preferred_element_type=jnp.float32) + b2[...]
    h_o[...] = h
    x = jnp.dot(h, w[...], preferred_element_type=jnp.float32)
    asd_o[...] = jnp.dot(x, a[...], preferred_element_type=jnp.float32)
    j = pl.program_id(1)
    x2_o[...] = jnp.where(j == 0, x[:, :32], x[:, 32:])


def _encproj(nf_pad, w1, b1, w2, b2, w, a):
    return pl.pallas_call(
        _encproj_body,
        grid=(NB, 2),
        in_specs=[
            pl.BlockSpec((BR, NFEAT), lambda i, j: (i, 0)),
            pl.BlockSpec((NFEAT, HID), lambda i, j: (0, 0)),
            pl.BlockSpec((1, HID), lambda i, j: (0, 0)),
            pl.BlockSpec((HID, HID), lambda i, j: (0, 0)),
            pl.BlockSpec((1, HID), lambda i, j: (0, 0)),
            pl.BlockSpec((HID, HID), lambda i, j: (0, 0)),
            pl.BlockSpec((HID, 2 * HEADS), lambda i, j: (0, 0)),
        ],
        out_specs=[
            pl.BlockSpec((BR, HID), lambda i, j: (i, 0)),
            pl.BlockSpec((BR, 32), lambda i, j: (j * NB + i, 0)),
            pl.BlockSpec((BR, 2 * HEADS), lambda i, j: (i, 0)),
        ],
        out_shape=[
            jax.ShapeDtypeStruct((NP, HID), jnp.float32),
            jax.ShapeDtypeStruct((2 * NP, 32), jnp.float32),
            jax.ShapeDtypeStruct((NP, 2 * HEADS), jnp.float32),
        ],
    )(nf_pad, w1, b1, w2, b2, w, a)


def _postproj_body(hres, olo, ohi, d0, d1, sel, bias, g, b, w, a,
                   h_o, x2_o, asd_o):
    den = d0[...][0][:, :HEADS] + d1[...][0][:, :HEADS]
    inv = 1.0 / (den + 1e-16)
    inv64 = jnp.dot(inv, sel[...], preferred_element_type=jnp.float32)
    agg = jnp.concatenate([olo[...][0], ohi[...][0]], axis=-1)
    y = agg * inv64 + bias[...] + hres[...]
    m = jnp.mean(y, axis=-1, keepdims=True)
    yc = y - m
    var = jnp.mean(yc * yc, axis=-1, keepdims=True)
    h = jnp.maximum(yc * lax.rsqrt(var + 1e-5) * g[...] + b[...], 0.0)
    h_o[...] = h
    x = jnp.dot(h, w[...], preferred_element_type=jnp.float32)
    asd_o[...] = jnp.dot(x, a[...], preferred_element_type=jnp.float32)
    j = pl.program_id(1)
    x2_o[...] = jnp.where(j == 0, x[:, :32], x[:, 32:])


def _postproj(hres, out2, denp, sel, bias, g, b, w, a):
    return pl.pallas_call(
        _postproj_body,
        grid=(NB, 2),
        in_specs=[
            pl.BlockSpec((BR, HID), lambda i, j: (i, 0)),
            pl.BlockSpec((1, BR, 32), lambda i, j: (0, i, 0)),
            pl.BlockSpec((1, BR, 32), lambda i, j: (1, i, 0)),
            pl.BlockSpec((1, BR, 8), lambda i, j: (0, i, 0)),
            pl.BlockSpec((1, BR, 8), lambda i, j: (1, i, 0)),
            pl.BlockSpec((HEADS, HID), lambda i, j: (0, 0)),
            pl.BlockSpec((1, HID), lambda i, j: (0, 0)),
            pl.BlockSpec((1, HID), lambda i, j: (0, 0)),
            pl.BlockSpec((1, HID), lambda i, j: (0, 0)),
            pl.BlockSpec((HID, HID), lambda i, j: (0, 0)),
            pl.BlockSpec((HID, 2 * HEADS), lambda i, j: (0, 0)),
        ],
        out_specs=[
            pl.BlockSpec((BR, HID), lambda i, j: (i, 0)),
            pl.BlockSpec((BR, 32), lambda i, j: (j * NB + i, 0)),
            pl.BlockSpec((BR, 2 * HEADS), lambda i, j: (i, 0)),
        ],
        out_shape=[
            jax.ShapeDtypeStruct((NP, HID), jnp.float32),
            jax.ShapeDtypeStruct((2 * NP, 32), jnp.float32),
            jax.ShapeDtypeStruct((NP, 2 * HEADS), jnp.float32),
        ],
    )(hres, out2, out2, denp, denp, sel, bias, g, b, w, a)


def _post_body(hres, olo, ohi, d0, d1, sel, bias, g, b, out):
    den = d0[...][0][:, :HEADS] + d1[...][0][:, :HEADS]
    inv = 1.0 / (den + 1e-16)
    inv64 = jnp.dot(inv, sel[...], preferred_element_type=jnp.float32)
    agg = jnp.concatenate([olo[...][0], ohi[...][0]], axis=-1)
    y = agg * inv64 + bias[...] + hres[...]
    m = jnp.mean(y, axis=-1, keepdims=True)
    yc = y - m
    var = jnp.mean(yc * yc, axis=-1, keepdims=True)
    out[...] = jnp.maximum(yc * lax.rsqrt(var + 1e-5) * g[...] + b[...], 0.0)


def _postprocess(hres, out2, denp, sel, bias, g, b):
    return pl.pallas_call(
        _post_body,
        grid=(NB,),
        in_specs=[
            pl.BlockSpec((BR, HID), lambda i: (i, 0)),
            pl.BlockSpec((1, BR, 32), lambda i: (0, i, 0)),
            pl.BlockSpec((1, BR, 32), lambda i: (1, i, 0)),
            pl.BlockSpec((1, BR, 8), lambda i: (0, i, 0)),
            pl.BlockSpec((1, BR, 8), lambda i: (1, i, 0)),
            pl.BlockSpec((HEADS, HID), lambda i: (0, 0)),
            pl.BlockSpec((1, HID), lambda i: (0, 0)),
            pl.BlockSpec((1, HID), lambda i: (0, 0)),
            pl.BlockSpec((1, HID), lambda i: (0, 0)),
        ],
        out_specs=pl.BlockSpec((BR, HID), lambda i: (i, 0)),
        out_shape=jax.ShapeDtypeStruct((NP, HID), jnp.float32),
    )(hres, out2, out2, denp, denp, sel, bias, g, b)


def _mlp_body(pf, w1, b1, w2, b2, out):
    h1 = jnp.maximum(jnp.dot(pf[...], w1[...], preferred_element_type=jnp.float32)
                     + b1[...], 0.0)
    out[...] = jnp.dot(h1, w2[...], preferred_element_type=jnp.float32) + b2[...]


def _pipeline_mlp(pf_pad, w1p, b1, w2, b2):
    return pl.pallas_call(
        _mlp_body,
        grid=(1,),
        in_specs=[
            pl.BlockSpec((8, 16), lambda i: (0, 0)),
            pl.BlockSpec((16, HID), lambda i: (0, 0)),
            pl.BlockSpec((1, HID), lambda i: (0, 0)),
            pl.BlockSpec((HID, HID), lambda i: (0, 0)),
            pl.BlockSpec((1, HID), lambda i: (0, 0)),
        ],
        out_specs=pl.BlockSpec((8, HID), lambda i: (0, 0)),
        out_shape=jax.ShapeDtypeStruct((8, HID), jnp.float32),
    )(pf_pad, w1p, b1, w2, b2)


GB = 4
NROW = EPAD // CH
NG_E = EPAD // 16 // (GB * CH)
GBD = 16
NG_D = EPAD // 32 // (GBD * CH)


GBP = 2
NG_P = EPAD // 16 // (GBP * CH)


def _sc_edge_body(src2_h, dst2_h, asd_h, x2_h, z32_h, out2_h,
                  idxs2, idxx2, idxd2, bufs3, bufd3, xbuf3,
                  out_sh, sems0, sems1, semx0, semx1, semo0, semo1):
    c = lax.axis_index("c")
    s = lax.axis_index("s")
    lo = s * NSTRIPE
    pltpu.sync_copy(z32_h.at[pl.ds(lo, NSTRIPE)], out_sh.at[pl.ds(lo, NSTRIPE)])
    plsc.subcore_barrier()

    iota = lax.broadcasted_iota(jnp.int32, (16,), 0)
    rbase = s * (NROW // 16)
    xoff = c * NP
    cb = 2 * c
    cbv = jnp.broadcast_to(cb, (16,))
    cbv1 = jnp.broadcast_to(cb + 1, (16,))
    sems = (sems0, sems1)
    semx = (semx0, semx1)
    semo = (semo0, semo1)

    def issue(g, p):
        r0 = rbase + g * GBP
        pltpu.sync_copy(src2_h.at[pl.ds(r0, GBP)], idxs2.at[p])
        pltpu.sync_copy(dst2_h.at[pl.ds(r0, GBP)], idxd2.at[p])
        for b in range(GBP):
            pltpu.async_copy(asd_h.at[idxs2.at[p, b]], bufs3.at[p, b], sems[p])
            pltpu.async_copy(asd_h.at[idxd2.at[p, b]], bufd3.at[p, b], sems[p])
        for b in range(GBP):
            def adj(k, cr, p=p, b=b):
                sl = pl.ds(k * 16, 16)
                idxx2[p, b, sl] = idxs2[p, b, sl] + xoff
                return cr
            lax.fori_loop(0, CH // 16, adj, 0, unroll=8)
        for b in range(GBP):
            pltpu.async_copy(x2_h.at[idxx2.at[p, b]], xbuf3.at[p, b], semx[p])

    def drain_asd(p):
        for _ in range(2 * GBP):
            pltpu.make_async_copy(asd_h.at[pl.ds(0, CH)],
                                  bufs3.at[p, 0], sems[p]).wait()

    def drain_x(p):
        for _ in range(GBP):
            pltpu.make_async_copy(x2_h.at[pl.ds(0, CH)],
                                  xbuf3.at[p, 0], semx[p]).wait()

    def drain_out(p):
        for _ in range(GBP):
            pltpu.make_async_copy(x2_h.at[pl.ds(0, CH)],
                                  xbuf3.at[p, 0], semo[p]).wait()

    def compute(p, mid=None):
        drain_asd(p)
        pv = jnp.broadcast_to(p, (16,))
        for b in range(GBP):
            bv = jnp.broadcast_to(b, (16,))

            def lane(j, cr, bv=bv):
                pp = j * 16 + iota
                r = pp // 4
                col = pp % 4
                vs = plsc.load_gather(bufs3, [pv, bv, r, col])
                vd = plsc.load_gather(bufd3, [pv, bv, r, col + 4])
                al = vs + vd
                al = jnp.where(al >= 0.0, al, al * 0.2)
                plsc.store_scatter(bufd3, [pv, bv, r, col], jnp.exp(al))
                return cr

            lax.fori_loop(0, (CH * 4) // 16, lane, 0, unroll=8)
        if mid is not None:
            mid()
        drain_x(p)
        for b in range(GBP):
            bv = jnp.broadcast_to(b, (16,))

            def edge(e, cr, bv=bv, b=b):
                ev = jnp.broadcast_to(e, (16,))
                c0 = plsc.load_gather(bufd3, [pv, bv, ev, cbv])
                c1 = plsc.load_gather(bufd3, [pv, bv, ev, cbv1])
                xbuf3[p, b, e, pl.ds(0, 16)] = xbuf3[p, b, e, pl.ds(0, 16)] * c0
                xbuf3[p, b, e, pl.ds(16, 16)] = xbuf3[p, b, e, pl.ds(16, 16)] * c1
                return cr

            lax.fori_loop(0, CH, edge, 0, unroll=8)
        for b in range(GBP):
            pltpu.async_copy(xbuf3.at[p, b], out_sh.at[idxd2.at[p, b]],
                             semo[p], add=True)

    issue(0, 0)
    NPAIR = NG_P // 2

    def pair(g2, carry):
        g = 2 * g2

        @pl.when(g2 > 0)
        def _():
            drain_out(1)
        issue(g + 1, 1)

        def mid0():
            pass

        compute(0, mid0)

        def mid1():
            @pl.when(g2 < NPAIR - 1)
            def _():
                drain_out(0)
                issue(g + 2, 0)

        compute(1, mid1)
        return carry

    lax.fori_loop(0, NPAIR, pair, 0)
    drain_out(0)
    drain_out(1)
    plsc.subcore_barrier()
    pltpu.sync_copy(out_sh.at[pl.ds(lo, NSTRIPE)],
                    out2_h.at[c, pl.ds(lo, NSTRIPE)])


@functools.partial(
    pl.kernel,
    mesh=plsc.VectorSubcoreMesh(core_axis_name="c", subcore_axis_name="s"),
    compiler_params=pltpu.CompilerParams(
        use_tc_tiling_on_sc=False, needs_layout_passes=False),
    out_type=[
        jax.ShapeDtypeStruct((2, NP, 32), jnp.float32),
    ],
    scratch_types=[
        pltpu.VMEM((2, GBP, CH), jnp.int32),
        pltpu.VMEM((2, GBP, CH), jnp.int32),
        pltpu.VMEM((2, GBP, CH), jnp.int32),
        pltpu.VMEM((2, GBP, CH, 2 * HEADS), jnp.float32),
        pltpu.VMEM((2, GBP, CH, 2 * HEADS), jnp.float32),
        pltpu.VMEM((2, GBP, CH, 32), jnp.float32),
        pltpu.VMEM_SHARED((NP, 32), jnp.float32),
        pltpu.SemaphoreType.DMA,
        pltpu.SemaphoreType.DMA,
        pltpu.SemaphoreType.DMA,
        pltpu.SemaphoreType.DMA,
        pltpu.SemaphoreType.DMA,
        pltpu.SemaphoreType.DMA,
    ],
)
def _sc_edge(src2_h, dst2_h, asd_h, x2_h, z32_h, out2_h,
             idxs2, idxx2, idxd2, bufs3, bufd3, xbuf3,
             out_sh, sems0, sems1, semx0, semx1, semo0, semo1):
    _sc_edge_body(src2_h, dst2_h, asd_h, x2_h, z32_h, out2_h,
                  idxs2, idxx2, idxd2, bufs3, bufd3, xbuf3,
                  out_sh, sems0, sems1, semx0, semx1, semo0, semo1)


GBD = 8
NG_D = EPAD // 32 // (GBD * CH)


def _sc_den_body(src2_h, dst2_h, asd_h, z8_h, den_h,
                 idxs2, idxd2, bufs3, bufd3, exb3,
                 den_sh, sems0, sems1, semo0, semo1):
    c = lax.axis_index("c")
    s = lax.axis_index("s")
    lo = s * NSTRIPE
    pltpu.sync_copy(z8_h.at[pl.ds(lo, NSTRIPE)], den_sh.at[pl.ds(lo, NSTRIPE)])
    plsc.subcore_barrier()
    iota = lax.broadcasted_iota(jnp.int32, (16,), 0)
    rbase = (c * 16 + s) * (NROW // 32)
    sems = (sems0, sems1)
    semo = (semo0, semo1)

    def issue(g, p):
        r0 = rbase + g * GBD
        pltpu.sync_copy(src2_h.at[pl.ds(r0, GBD)], idxs2.at[p])
        pltpu.sync_copy(dst2_h.at[pl.ds(r0, GBD)], idxd2.at[p])
        for b in range(GBD):
            pltpu.async_copy(asd_h.at[idxs2.at[p, b]], bufs3.at[p, b], sems[p])
            pltpu.async_copy(asd_h.at[idxd2.at[p, b]], bufd3.at[p, b], sems[p])

    def drain_asd(p):
        for _ in range(2 * GBD):
            pltpu.make_async_copy(asd_h.at[pl.ds(0, CH)],
                                  bufs3.at[p, 0], sems[p]).wait()

    def drain_out(p):
        for _ in range(GBD):
            pltpu.make_async_copy(asd_h.at[pl.ds(0, CH)],
                                  exb3.at[p, 0], semo[p]).wait()

    def compute(p, mid=None):
        drain_asd(p)
        pv = jnp.broadcast_to(p, (16,))
        for b in range(GBD):
            bv = jnp.broadcast_to(b, (16,))

            def lane(j, cr, bv=bv):
                pp = j * 16 + iota
                r = pp // 4
                col = pp % 4
                vs = plsc.load_gather(bufs3, [pv, bv, r, col])
                vd = plsc.load_gather(bufd3, [pv, bv, r, col + 4])
                al = vs + vd
                al = jnp.where(al >= 0.0, al, al * 0.2)
                ev = jnp.exp(al)
                plsc.store_scatter(exb3, [pv, bv, r, col], ev)
                plsc.store_scatter(exb3, [pv, bv, r, col + 4], ev)
                return cr

            lax.fori_loop(0, (CH * 4) // 16, lane, 0, unroll=8)
            if b == 0 and mid is not None:
                mid()
        for b in range(GBD):
            pltpu.async_copy(exb3.at[p, b], den_sh.at[idxd2.at[p, b]],
                             semo[p], add=True)

    issue(0, 0)
    NPAIR = NG_D // 2

    def pair(g2, carry):
        g = 2 * g2

        @pl.when(g2 > 0)
        def _():
            drain_out(1)
        issue(g + 1, 1)
        compute(0)

        def mid1():
            @pl.when(g2 < NPAIR - 1)
            def _():
                drain_out(0)
                issue(g + 2, 0)

        compute(1, mid1)
        return carry

    lax.fori_loop(0, NPAIR, pair, 0)
    drain_out(0)
    drain_out(1)
    plsc.subcore_barrier()
    pltpu.sync_copy(den_sh.at[pl.ds(lo, NSTRIPE)],
                    den_h.at[c, pl.ds(lo, NSTRIPE)])


@functools.partial(
    pl.kernel,
    mesh=plsc.VectorSubcoreMesh(core_axis_name="c", subcore_axis_name="s"),
    compiler_params=pltpu.CompilerParams(
        use_tc_tiling_on_sc=False, needs_layout_passes=False),
    out_type=[jax.ShapeDtypeStruct((2, NP, 8), jnp.float32)],
    scratch_types=[
        pltpu.VMEM((2, GBD, CH), jnp.int32),
        pltpu.VMEM((2, GBD, CH), jnp.int32),
        pltpu.VMEM((2, GBD, CH, 2 * HEADS), jnp.float32),
        pltpu.VMEM((2, GBD, CH, 2 * HEADS), jnp.float32),
        pltpu.VMEM((2, GBD, CH, 8), jnp.float32),
        pltpu.VMEM_SHARED((NP, 8), jnp.float32),
        pltpu.SemaphoreType.DMA,
        pltpu.SemaphoreType.DMA,
        pltpu.SemaphoreType.DMA,
        pltpu.SemaphoreType.DMA,
    ],
)
def _sc_den(src2_h, dst2_h, asd_h, z8_h, den_h,
            idxs2, idxd2, bufs3, bufd3, exb3,
            den_sh, sems0, sems1, semo0, semo1):
    _sc_den_body(src2_h, dst2_h, asd_h, z8_h, den_h,
                 idxs2, idxd2, bufs3, bufd3, exb3,
                 den_sh, sems0, sems1, semo0, semo1)


def kernel(node_features, edge_index, edge_attr, pipeline_state,
           register_pressure, ready_mask, scheduled_mask,
           enc_W1, enc_b1, enc_W2, enc_b2,
           gat_W, gat_att_src, gat_att_dst, gat_bias,
           ln_g, ln_b,
           pip_W1, pip_b1, pip_W2, pip_b2):
    f32 = jnp.float32
    nf_pad = jnp.pad(node_features, ((0, NP - N), (0, 0)))

    loops = jnp.arange(N, dtype=edge_index.dtype)
    pad_e = jnp.full((EPAD - E - N,), DUMMY, dtype=edge_index.dtype)
    src = jnp.concatenate([edge_index[0], loops, pad_e]).reshape(NROW, CH)
    dst = jnp.concatenate([edge_index[1], loops, pad_e]).reshape(NROW, CH)

    z8 = jnp.zeros((NP, 8), f32)
    z32 = jnp.zeros((NP, 32), f32)
    sel = jnp.asarray(np.kron(np.eye(HEADS), np.ones((1, FH))), f32)
    eye = jnp.asarray(np.eye(HEADS), f32)

    def acat_i(i):
        a_src = (gat_att_src[i][:, :, None] * eye[:, None, :]).reshape(HID, HEADS)
        a_dst = (gat_att_dst[i][:, :, None] * eye[:, None, :]).reshape(HID, HEADS)
        return jnp.concatenate([a_src, a_dst], axis=1)

    h, x2, asd = _encproj(nf_pad, enc_W1, enc_b1.reshape(1, HID),
                          enc_W2, enc_b2.reshape(1, HID), gat_W[0], acat_i(0))
    for i in range(NLAYERS):
        out2 = _sc_edge(src, dst, asd, x2, z32)[0]
        denp = _sc_den(src, dst, asd, z8)[0]
        if i < NLAYERS - 1:
            h, x2, asd = _postproj(h, out2, denp, sel,
                                   gat_bias[i].reshape(1, HID),
                                   ln_g[i].reshape(1, HID), ln_b[i].reshape(1, HID),
                                   gat_W[i + 1], acat_i(i + 1))
        else:
            h = _postprocess(h, out2, denp, sel,
                             gat_bias[i].reshape(1, HID),
                             ln_g[i].reshape(1, HID), ln_b[i].reshape(1, HID))

    pf = jnp.concatenate([pipeline_state, register_pressure])
    pf_pad = jnp.zeros((8, 16), f32).at[0, :9].set(pf)
    w1p = jnp.pad(pip_W1, ((0, 16 - 9), (0, 0)))
    q = _pipeline_mlp(pf_pad, w1p, pip_b1.reshape(1, HID),
                      pip_W2, pip_b2.reshape(1, HID))
    return (h[:N], q[0])
